# Initial kernel scaffold; baseline (speedup 1.0000x reference)
import jax
import jax.numpy as jnp
from jax.experimental import pallas as pl


def kernel(positions, shifts, embed_W, radial_W0, radial_W1, msg_W0, msg_W1, ro0_W, ro1_W1, ro1_W2, scale, shift, species, edge_index):
    raise NotImplementedError("write your pallas kernel here")


# speedup vs baseline: 2.0951x; 2.0951x over previous
import jax
import jax.numpy as jnp
from jax import lax
from jax.experimental import pallas as pl
from jax.experimental.pallas import tpu as pltpu
from jax.experimental.pallas import tpu_sc as plsc

N = 10000
E = 320000
D = 128
DH = D // 2
NB = 8
RC = 5.0
MLP_H = 64
SILU_SCALE = 1.6765324703310907
PI = 3.141592653589793

NCORE = 2
NSUB = 16
NWORK = NCORE * NSUB
EPW = E // NWORK
EPT = E // NSUB
WB = 80
NWIN = EPT // WB
NPAD = 10240
ROWS_PER_SUB = NPAD // NSUB
ZR = 128
NZC = ROWS_PER_SUB // ZR

_INTERPRET = False


def _sc_mesh():
    return plsc.VectorSubcoreMesh(core_axis_name="c", subcore_axis_name="s",
                                  num_cores=NCORE, num_subcores=NSUB)


def _geom_body(px_hbm, py_hbm, pz_hbm, sx_hbm, sy_hbm, sz_hbm,
               ii_hbm, jj_hbm, d2_out,
               px, py, pz, sx, sy, sz, iv, jv, d2v):
    cid = lax.axis_index("c")
    sid = lax.axis_index("s")
    wid = sid * NCORE + cid
    base = wid * EPW
    pltpu.sync_copy(px_hbm, px)
    pltpu.sync_copy(py_hbm, py)
    pltpu.sync_copy(pz_hbm, pz)
    pltpu.sync_copy(sx_hbm.at[pl.ds(base, EPW)], sx)
    pltpu.sync_copy(sy_hbm.at[pl.ds(base, EPW)], sy)
    pltpu.sync_copy(sz_hbm.at[pl.ds(base, EPW)], sz)
    pltpu.sync_copy(ii_hbm.at[pl.ds(base, EPW)], iv)
    pltpu.sync_copy(jj_hbm.at[pl.ds(base, EPW)], jv)

    def grp(g, carry):
        s = pl.ds(g * 16, 16)
        a = iv[s]
        b = jv[s]
        dx = plsc.load_gather(px, [a]) - plsc.load_gather(px, [b]) - sx[s]
        dy = plsc.load_gather(py, [a]) - plsc.load_gather(py, [b]) - sy[s]
        dz = plsc.load_gather(pz, [a]) - plsc.load_gather(pz, [b]) - sz[s]
        d2v[s] = dx * dx + dy * dy + dz * dz
        return carry

    lax.fori_loop(0, EPW // 16, grp, 0)
    pltpu.sync_copy(d2v, d2_out.at[pl.ds(base, EPW)])


def _edge_d2(pxyz, sxyz, idx_i, idx_j):
    return pl.kernel(
        _geom_body,
        out_type=jax.ShapeDtypeStruct((E,), jnp.float32),
        mesh=_sc_mesh(),
        scratch_types=[
            pltpu.VMEM((N,), jnp.float32),
            pltpu.VMEM((N,), jnp.float32),
            pltpu.VMEM((N,), jnp.float32),
            pltpu.VMEM((EPW,), jnp.float32),
            pltpu.VMEM((EPW,), jnp.float32),
            pltpu.VMEM((EPW,), jnp.float32),
            pltpu.VMEM((EPW,), jnp.int32),
            pltpu.VMEM((EPW,), jnp.int32),
            pltpu.VMEM((EPW,), jnp.float32),
        ],
        compiler_params=pltpu.CompilerParams(needs_layout_passes=False, use_tc_tiling_on_sc=False),
        interpret=_INTERPRET,
    )(pxyz[0], pxyz[1], pxyz[2], sxyz[0], sxyz[1], sxyz[2], idx_i, idx_j)


def _msgagg_body(h2flat, w2_hbm, iiw, jjw, agg_out,
                 ii_v, jj_v, jadj, wbuf, hbuf, mbuf, zbuf, agg_sh):
    cid = lax.axis_index("c")
    sid = lax.axis_index("s")

    def zrow(e, carry):
        for c in range(DH // 16):
            zbuf[e, pl.ds(c * 16, 16)] = jnp.zeros((16,), jnp.float32)
        return carry

    lax.fori_loop(0, ZR, zrow, 0)
    for k in range(NZC):
        pltpu.sync_copy(zbuf, agg_sh.at[pl.ds(sid * ROWS_PER_SUB + k * ZR, ZR)])
    plsc.subcore_barrier()

    pltpu.sync_copy(iiw.at[sid], ii_v)
    pltpu.sync_copy(jjw.at[sid], jj_v)

    def win(w, carry):
        eoff = (sid * NWIN + w) * WB
        pltpu.sync_copy(w2_hbm.at[cid, pl.ds(eoff, WB)], wbuf)
        def jrow(q, c2):
            s = pl.ds(q * 16, 16)
            jadj[0, s] = jj_v[w, s] + cid * N
            return c2
        lax.fori_loop(0, WB // 16, jrow, 0)
        pltpu.sync_copy(h2flat.at[jadj.at[0]], hbuf)

        def erow(e, c2):
            for c in range(DH // 16):
                s = pl.ds(c * 16, 16)
                mbuf[e, s] = wbuf[e, s] * hbuf[e, s]
            return c2

        lax.fori_loop(0, WB, erow, 0)
        pltpu.sync_copy(mbuf, agg_sh.at[ii_v.at[w]], add=True)
        return carry

    lax.fori_loop(0, NWIN, win, 0)
    plsc.subcore_barrier()

    for k in range(NZC):
        r0 = sid * ROWS_PER_SUB + k * ZR
        pltpu.sync_copy(agg_sh.at[pl.ds(r0, ZR)], zbuf)
        pltpu.sync_copy(zbuf, agg_out.at[cid, pl.ds(r0, ZR)])


def _msg_aggregate(h2flat, w2, iiw, jjw):
    return pl.kernel(
        _msgagg_body,
        out_type=jax.ShapeDtypeStruct((NCORE, NPAD, DH), jnp.float32),
        mesh=_sc_mesh(),
        scratch_types=[
            pltpu.VMEM((NWIN, WB), jnp.int32),
            pltpu.VMEM((NWIN, WB), jnp.int32),
            pltpu.VMEM((1, WB), jnp.int32),
            pltpu.VMEM((WB, DH), jnp.float32),
            pltpu.VMEM((WB, DH), jnp.float32),
            pltpu.VMEM((WB, DH), jnp.float32),
            pltpu.VMEM((ZR, DH), jnp.float32),
            pltpu.VMEM_SHARED((NPAD, DH), jnp.float32),
        ],
        compiler_params=pltpu.CompilerParams(needs_layout_passes=False, use_tc_tiling_on_sc=False),
        interpret=_INTERPRET,
    )(h2flat, w2, iiw, jjw)


BLK_E = 2000
BLK_N = 2000


def _rbfw_body(d2_ref, rw0_ref, rw1_ref, w0_ref, w1_ref):
    d2 = d2_ref[...]
    r = jnp.sqrt(d2) + 1e-9
    nvec = (lax.broadcasted_iota(jnp.int32, (1, NB), 1).astype(jnp.float32)
            + 1.0) * (PI / RC)
    rbf = jnp.sin(r * nvec) / r
    w0 = jnp.dot(rbf, rw0_ref[...], preferred_element_type=jnp.float32,
                 precision=lax.Precision.HIGHEST)
    w1 = jnp.dot(rbf, rw1_ref[...], preferred_element_type=jnp.float32,
                 precision=lax.Precision.HIGHEST)
    w0_ref[0] = w0[:, :DH]
    w0_ref[1] = w0[:, DH:]
    w1_ref[0] = w1[:, :DH]
    w1_ref[1] = w1[:, DH:]


def _radial_w(d2col, rW0, rW1):
    grid = E // BLK_E
    return pl.pallas_call(
        _rbfw_body,
        grid=(grid,),
        in_specs=[
            pl.BlockSpec((BLK_E, 1), lambda i: (i, 0)),
            pl.BlockSpec((NB, D), lambda i: (0, 0)),
            pl.BlockSpec((NB, D), lambda i: (0, 0)),
        ],
        out_specs=[
            pl.BlockSpec((2, BLK_E, DH), lambda i: (0, i, 0)),
            pl.BlockSpec((2, BLK_E, DH), lambda i: (0, i, 0)),
        ],
        out_shape=[
            jax.ShapeDtypeStruct((2, E, DH), jnp.float32),
            jax.ShapeDtypeStruct((2, E, DH), jnp.float32),
        ],
        interpret=_INTERPRET,
    )(d2col, rW0, rW1)


def _embed_body(spc_ref, ew_ref, scrow_ref, shrow_ref, h2_ref, scl_ref, shf_ref):
    spc = spc_ref[...]
    oh = (spc == lax.broadcasted_iota(jnp.int32, (1, 16), 1)).astype(jnp.float32)
    h = jnp.dot(oh, ew_ref[...], preferred_element_type=jnp.float32,
                 precision=lax.Precision.HIGHEST)
    h2_ref[0] = h[:, :DH]
    h2_ref[1] = h[:, DH:]
    scl_ref[...] = jnp.sum(oh * scrow_ref[...], axis=1, keepdims=True)
    shf_ref[...] = jnp.sum(oh * shrow_ref[...], axis=1, keepdims=True)


def _embed(spccol, ew_pad, scrow, shrow):
    grid = N // BLK_N
    return pl.pallas_call(
        _embed_body,
        grid=(grid,),
        in_specs=[
            pl.BlockSpec((BLK_N, 1), lambda i: (i, 0)),
            pl.BlockSpec((16, D), lambda i: (0, 0)),
            pl.BlockSpec((1, 16), lambda i: (0, 0)),
            pl.BlockSpec((1, 16), lambda i: (0, 0)),
        ],
        out_specs=[
            pl.BlockSpec((2, BLK_N, DH), lambda i: (0, i, 0)),
            pl.BlockSpec((BLK_N, 1), lambda i: (i, 0)),
            pl.BlockSpec((BLK_N, 1), lambda i: (i, 0)),
        ],
        out_shape=[
            jax.ShapeDtypeStruct((2, N, DH), jnp.float32),
            jax.ShapeDtypeStruct((N, 1), jnp.float32),
            jax.ShapeDtypeStruct((N, 1), jnp.float32),
        ],
        interpret=_INTERPRET,
    )(spccol, ew_pad, scrow, shrow)


def _upd_body(h2_ref, a_ref, mw_ref, out_ref):
    h = jnp.concatenate((h2_ref[0], h2_ref[1]), axis=1)
    acc = jnp.concatenate((a_ref[0], a_ref[1]), axis=1)
    hn = h + jnp.dot(acc, mw_ref[...], preferred_element_type=jnp.float32,
                 precision=lax.Precision.HIGHEST)
    out_ref[0] = hn[:, :DH]
    out_ref[1] = hn[:, DH:]


def _update(h2, agg, mW):
    grid = N // BLK_N
    return pl.pallas_call(
        _upd_body,
        grid=(grid,),
        in_specs=[
            pl.BlockSpec((2, BLK_N, DH), lambda i: (0, i, 0)),
            pl.BlockSpec((2, BLK_N, DH), lambda i: (0, i, 0)),
            pl.BlockSpec((D, D), lambda i: (0, 0)),
        ],
        out_specs=pl.BlockSpec((2, BLK_N, DH), lambda i: (0, i, 0)),
        out_shape=jax.ShapeDtypeStruct((2, N, DH), jnp.float32),
        interpret=_INTERPRET,
    )(h2, agg, mW)


def _final_body(h2_ref, a_ref, mw_ref, ro0r_ref, w1p_ref, w2r_ref,
                scl_ref, shf_ref, en_ref):
    h1 = jnp.concatenate((h2_ref[0], h2_ref[1]), axis=1)
    acc = jnp.concatenate((a_ref[0], a_ref[1]), axis=1)
    h2 = h1 + jnp.dot(acc, mw_ref[...], preferred_element_type=jnp.float32,
                 precision=lax.Precision.HIGHEST)
    out0 = jnp.sum(h1 * ro0r_ref[...], axis=1, keepdims=True)
    t = jnp.dot(h2, w1p_ref[...], preferred_element_type=jnp.float32,
                 precision=lax.Precision.HIGHEST)
    t = (t * jax.nn.sigmoid(t)) * SILU_SCALE
    out1 = jnp.sum(t * w2r_ref[...], axis=1, keepdims=True)
    en_ref[...] = scl_ref[...] * (out0 + out1) + shf_ref[...]


def _final(h2, agg, mW1, ro0row, w1pad, w2row, sclcol, shfcol):
    grid = N // BLK_N
    return pl.pallas_call(
        _final_body,
        grid=(grid,),
        in_specs=[
            pl.BlockSpec((2, BLK_N, DH), lambda i: (0, i, 0)),
            pl.BlockSpec((2, BLK_N, DH), lambda i: (0, i, 0)),
            pl.BlockSpec((D, D), lambda i: (0, 0)),
            pl.BlockSpec((1, D), lambda i: (0, 0)),
            pl.BlockSpec((D, D), lambda i: (0, 0)),
            pl.BlockSpec((1, D), lambda i: (0, 0)),
            pl.BlockSpec((BLK_N, 1), lambda i: (i, 0)),
            pl.BlockSpec((BLK_N, 1), lambda i: (i, 0)),
        ],
        out_specs=pl.BlockSpec((BLK_N, 1), lambda i: (i, 0)),
        out_shape=jax.ShapeDtypeStruct((N, 1), jnp.float32),
        interpret=_INTERPRET,
    )(h2, agg, mW1, ro0row, w1pad, w2row, sclcol, shfcol)


def kernel(positions, shifts, embed_W, radial_W0, radial_W1, msg_W0, msg_W1,
           ro0_W, ro1_W1, ro1_W2, scale, shift, species, edge_index):
    idx_i = edge_index[0]
    idx_j = edge_index[1]
    pxyz = [positions[:, k] for k in range(3)]
    sxyz = [shifts[:, k] for k in range(3)]
    iiw = idx_i.reshape(NSUB, NWIN, WB)
    jjw = idx_j.reshape(NSUB, NWIN, WB)

    d2 = _edge_d2(pxyz, sxyz, idx_i, idx_j)
    w0, w1 = _radial_w(d2.reshape(E, 1), radial_W0, radial_W1)

    ew_pad = jnp.pad(embed_W, ((0, 16 - embed_W.shape[0]), (0, 0)))
    scrow = jnp.pad(scale, (0, 16 - scale.shape[0])).reshape(1, 16)
    shrow = jnp.pad(shift, (0, 16 - shift.shape[0])).reshape(1, 16)
    h0, sclcol, shfcol = _embed(species.reshape(N, 1), ew_pad, scrow, shrow)

    agg0 = _msg_aggregate(h0.reshape(2 * N, DH), w0, iiw, jjw)[:, :N]
    h1 = _update(h0, agg0, msg_W0)

    agg1 = _msg_aggregate(h1.reshape(2 * N, DH), w1, iiw, jjw)[:, :N]

    ro0row = ro0_W.reshape(1, D)
    w1pad = jnp.pad(ro1_W1, ((0, 0), (0, D - MLP_H)))
    w2row = jnp.pad(ro1_W2, ((0, D - MLP_H), (0, 0))).reshape(1, D)
    en = _final(h1, agg1, msg_W1, ro0row, w1pad, w2row, sclcol, shfcol)
    return en[:, 0]


# speedup vs baseline: 2.1582x; 1.0302x over previous
import jax
import jax.numpy as jnp
from jax import lax
from jax.experimental import pallas as pl
from jax.experimental.pallas import tpu as pltpu
from jax.experimental.pallas import tpu_sc as plsc

N = 10000
E = 320000
D = 128
DH = D // 2
NB = 8
RC = 5.0
MLP_H = 64
SILU_SCALE = 1.6765324703310907
PI = 3.141592653589793

NCORE = 2
NSUB = 16
NWORK = NCORE * NSUB
EPW = E // NWORK
EPT = E // NSUB
WB = 80
NWIN = EPT // WB
NPAD = 10240
ROWS_PER_SUB = NPAD // NSUB
ZR = 128
NZC = ROWS_PER_SUB // ZR

_INTERPRET = False


def _sc_mesh():
    return plsc.VectorSubcoreMesh(core_axis_name="c", subcore_axis_name="s",
                                  num_cores=NCORE, num_subcores=NSUB)


def _geom_body(px_hbm, py_hbm, pz_hbm, sx_hbm, sy_hbm, sz_hbm,
               ii_hbm, jj_hbm, d2_out,
               px, py, pz, sx, sy, sz, iv, jv, d2v):
    cid = lax.axis_index("c")
    sid = lax.axis_index("s")
    wid = sid * NCORE + cid
    base = wid * EPW
    pltpu.sync_copy(px_hbm, px)
    pltpu.sync_copy(py_hbm, py)
    pltpu.sync_copy(pz_hbm, pz)
    pltpu.sync_copy(sx_hbm.at[pl.ds(base, EPW)], sx)
    pltpu.sync_copy(sy_hbm.at[pl.ds(base, EPW)], sy)
    pltpu.sync_copy(sz_hbm.at[pl.ds(base, EPW)], sz)
    pltpu.sync_copy(ii_hbm.at[pl.ds(base, EPW)], iv)
    pltpu.sync_copy(jj_hbm.at[pl.ds(base, EPW)], jv)

    def grp(g, carry):
        s = pl.ds(g * 16, 16)
        a = iv[s]
        b = jv[s]
        dx = plsc.load_gather(px, [a]) - plsc.load_gather(px, [b]) - sx[s]
        dy = plsc.load_gather(py, [a]) - plsc.load_gather(py, [b]) - sy[s]
        dz = plsc.load_gather(pz, [a]) - plsc.load_gather(pz, [b]) - sz[s]
        d2v[s] = dx * dx + dy * dy + dz * dz
        return carry

    lax.fori_loop(0, EPW // 16, grp, 0)
    pltpu.sync_copy(d2v, d2_out.at[pl.ds(base, EPW)])


def _edge_d2(pxyz, sxyz, idx_i, idx_j):
    return pl.kernel(
        _geom_body,
        out_type=jax.ShapeDtypeStruct((E,), jnp.float32),
        mesh=_sc_mesh(),
        scratch_types=[
            pltpu.VMEM((N,), jnp.float32),
            pltpu.VMEM((N,), jnp.float32),
            pltpu.VMEM((N,), jnp.float32),
            pltpu.VMEM((EPW,), jnp.float32),
            pltpu.VMEM((EPW,), jnp.float32),
            pltpu.VMEM((EPW,), jnp.float32),
            pltpu.VMEM((EPW,), jnp.int32),
            pltpu.VMEM((EPW,), jnp.int32),
            pltpu.VMEM((EPW,), jnp.float32),
        ],
        compiler_params=pltpu.CompilerParams(needs_layout_passes=False,
                                             use_tc_tiling_on_sc=False),
        interpret=_INTERPRET,
    )(pxyz[0], pxyz[1], pxyz[2], sxyz[0], sxyz[1], sxyz[2], idx_i, idx_j)


def _msgagg_body(h_hbm, w_hbm, iiw, jjw, agg_out,
                 ii_v, jj_v, wbuf, hbuf, mbuf, zbuf, agg_sh):
    cid = lax.axis_index("c")
    sid = lax.axis_index("s")
    choff = cid * DH

    def zrow(e, carry):
        for c in range(DH // 16):
            zbuf[e, pl.ds(c * 16, 16)] = jnp.zeros((16,), jnp.float32)
        return carry

    lax.fori_loop(0, ZR, zrow, 0)
    for k in range(NZC):
        pltpu.sync_copy(zbuf, agg_sh.at[pl.ds(sid * ROWS_PER_SUB + k * ZR, ZR)])
    plsc.subcore_barrier()

    pltpu.sync_copy(iiw.at[sid], ii_v)
    pltpu.sync_copy(jjw.at[sid], jj_v)

    def win(w, carry):
        eoff = (sid * NWIN + w) * WB
        pltpu.sync_copy(w_hbm.at[pl.ds(eoff, WB)], wbuf)
        pltpu.sync_copy(h_hbm.at[jj_v.at[w]], hbuf)

        def erow(e, c2):
            for c in range(DH // 16):
                so = pl.ds(c * 16, 16)
                si = pl.ds(choff + c * 16, 16)
                mbuf[e, so] = wbuf[e, si] * hbuf[e, si]
            return c2

        lax.fori_loop(0, WB, erow, 0)
        pltpu.sync_copy(mbuf, agg_sh.at[ii_v.at[w]], add=True)
        return carry

    lax.fori_loop(0, NWIN, win, 0)
    plsc.subcore_barrier()

    for k in range(NZC):
        r0 = sid * ROWS_PER_SUB + k * ZR
        pltpu.sync_copy(agg_sh.at[pl.ds(r0, ZR)], zbuf)
        pltpu.sync_copy(zbuf, agg_out.at[cid, pl.ds(r0, ZR)])


def _msg_aggregate(h, w, iiw, jjw):
    return pl.kernel(
        _msgagg_body,
        out_type=jax.ShapeDtypeStruct((NCORE, NPAD, DH), jnp.float32),
        mesh=_sc_mesh(),
        scratch_types=[
            pltpu.VMEM((NWIN, WB), jnp.int32),
            pltpu.VMEM((NWIN, WB), jnp.int32),
            pltpu.VMEM((WB, D), jnp.float32),
            pltpu.VMEM((WB, D), jnp.float32),
            pltpu.VMEM((WB, DH), jnp.float32),
            pltpu.VMEM((ZR, DH), jnp.float32),
            pltpu.VMEM_SHARED((NPAD, DH), jnp.float32),
        ],
        compiler_params=pltpu.CompilerParams(needs_layout_passes=False,
                                             use_tc_tiling_on_sc=False),
        interpret=_INTERPRET,
    )(h, w, iiw, jjw)


BLK_E = 2000
BLK_N = 2000


def _rbfw_body(d2_ref, rw0_ref, rw1_ref, w0_ref, w1_ref):
    d2 = d2_ref[0]
    r = jnp.sqrt(d2) + 1e-9
    rinv = 1.0 / r
    nvec = (lax.broadcasted_iota(jnp.int32, (NB, 1), 0).astype(jnp.float32)
            + 1.0) * (PI / RC)
    rbf_t = jnp.sin(nvec * r) * rinv
    dn = (((0,), (0,)), ((), ()))
    w0_ref[...] = lax.dot_general(rbf_t, rw0_ref[...], dn,
                                  preferred_element_type=jnp.float32,
                                  precision=lax.Precision.HIGHEST)
    w1_ref[...] = lax.dot_general(rbf_t, rw1_ref[...], dn,
                                  preferred_element_type=jnp.float32,
                                  precision=lax.Precision.HIGHEST)


def _radial_w(d2m3, rW0, rW1):
    grid = E // BLK_E
    return pl.pallas_call(
        _rbfw_body,
        grid=(grid,),
        in_specs=[
            pl.BlockSpec((1, 1, BLK_E), lambda i: (i, 0, 0)),
            pl.BlockSpec((NB, D), lambda i: (0, 0)),
            pl.BlockSpec((NB, D), lambda i: (0, 0)),
        ],
        out_specs=[
            pl.BlockSpec((BLK_E, D), lambda i: (i, 0)),
            pl.BlockSpec((BLK_E, D), lambda i: (i, 0)),
        ],
        out_shape=[
            jax.ShapeDtypeStruct((E, D), jnp.float32),
            jax.ShapeDtypeStruct((E, D), jnp.float32),
        ],
        interpret=_INTERPRET,
    )(d2m3, rW0, rW1)


def _embed_body(spc_ref, ew_ref, scrow_ref, shrow_ref, h_ref, scl_ref, shf_ref):
    spc = spc_ref[...]
    oh = (spc == lax.broadcasted_iota(jnp.int32, (1, 16), 1)).astype(jnp.float32)
    h_ref[...] = jnp.dot(oh, ew_ref[...], preferred_element_type=jnp.float32,
                         precision=lax.Precision.HIGHEST)
    scl_ref[...] = jnp.sum(oh * scrow_ref[...], axis=1, keepdims=True)
    shf_ref[...] = jnp.sum(oh * shrow_ref[...], axis=1, keepdims=True)


def _embed(spccol, ew_pad, scrow, shrow):
    grid = N // BLK_N
    return pl.pallas_call(
        _embed_body,
        grid=(grid,),
        in_specs=[
            pl.BlockSpec((BLK_N, 1), lambda i: (i, 0)),
            pl.BlockSpec((16, D), lambda i: (0, 0)),
            pl.BlockSpec((1, 16), lambda i: (0, 0)),
            pl.BlockSpec((1, 16), lambda i: (0, 0)),
        ],
        out_specs=[
            pl.BlockSpec((BLK_N, D), lambda i: (i, 0)),
            pl.BlockSpec((BLK_N, 1), lambda i: (i, 0)),
            pl.BlockSpec((BLK_N, 1), lambda i: (i, 0)),
        ],
        out_shape=[
            jax.ShapeDtypeStruct((N, D), jnp.float32),
            jax.ShapeDtypeStruct((N, 1), jnp.float32),
            jax.ShapeDtypeStruct((N, 1), jnp.float32),
        ],
        interpret=_INTERPRET,
    )(spccol, ew_pad, scrow, shrow)


def _upd_body(h_ref, a_ref, mw_ref, out_ref):
    acc = jnp.concatenate((a_ref[0], a_ref[1]), axis=1)
    out_ref[...] = h_ref[...] + jnp.dot(acc, mw_ref[...],
                                        preferred_element_type=jnp.float32,
                                        precision=lax.Precision.HIGHEST)


def _update(h, agg, mW):
    grid = N // BLK_N
    return pl.pallas_call(
        _upd_body,
        grid=(grid,),
        in_specs=[
            pl.BlockSpec((BLK_N, D), lambda i: (i, 0)),
            pl.BlockSpec((2, BLK_N, DH), lambda i: (0, i, 0)),
            pl.BlockSpec((D, D), lambda i: (0, 0)),
        ],
        out_specs=pl.BlockSpec((BLK_N, D), lambda i: (i, 0)),
        out_shape=jax.ShapeDtypeStruct((N, D), jnp.float32),
        interpret=_INTERPRET,
    )(h, agg, mW)


def _final_body(h1_ref, a_ref, mw_ref, ro0r_ref, w1p_ref, w2r_ref,
                scl_ref, shf_ref, en_ref):
    h1 = h1_ref[...]
    acc = jnp.concatenate((a_ref[0], a_ref[1]), axis=1)
    h2 = h1 + jnp.dot(acc, mw_ref[...], preferred_element_type=jnp.float32,
                      precision=lax.Precision.HIGHEST)
    out0 = jnp.sum(h1 * ro0r_ref[...], axis=1, keepdims=True)
    t = jnp.dot(h2, w1p_ref[...], preferred_element_type=jnp.float32,
                precision=lax.Precision.HIGHEST)
    t = (t * jax.nn.sigmoid(t)) * SILU_SCALE
    out1 = jnp.sum(t * w2r_ref[...], axis=1, keepdims=True)
    en_ref[...] = scl_ref[...] * (out0 + out1) + shf_ref[...]


def _final(h1, agg, mW1, ro0row, w1pad, w2row, sclcol, shfcol):
    grid = N // BLK_N
    return pl.pallas_call(
        _final_body,
        grid=(grid,),
        in_specs=[
            pl.BlockSpec((BLK_N, D), lambda i: (i, 0)),
            pl.BlockSpec((2, BLK_N, DH), lambda i: (0, i, 0)),
            pl.BlockSpec((D, D), lambda i: (0, 0)),
            pl.BlockSpec((1, D), lambda i: (0, 0)),
            pl.BlockSpec((D, D), lambda i: (0, 0)),
            pl.BlockSpec((1, D), lambda i: (0, 0)),
            pl.BlockSpec((BLK_N, 1), lambda i: (i, 0)),
            pl.BlockSpec((BLK_N, 1), lambda i: (i, 0)),
        ],
        out_specs=pl.BlockSpec((BLK_N, 1), lambda i: (i, 0)),
        out_shape=jax.ShapeDtypeStruct((N, 1), jnp.float32),
        interpret=_INTERPRET,
    )(h1, agg, mW1, ro0row, w1pad, w2row, sclcol, shfcol)


def kernel(positions, shifts, embed_W, radial_W0, radial_W1, msg_W0, msg_W1,
           ro0_W, ro1_W1, ro1_W2, scale, shift, species, edge_index):
    idx_i = edge_index[0]
    idx_j = edge_index[1]
    pxyz = [positions[:, k] for k in range(3)]
    sxyz = [shifts[:, k] for k in range(3)]
    iiw = idx_i.reshape(NSUB, NWIN, WB)
    jjw = idx_j.reshape(NSUB, NWIN, WB)

    d2 = _edge_d2(pxyz, sxyz, idx_i, idx_j)
    w0, w1 = _radial_w(d2.reshape(E // BLK_E, 1, BLK_E), radial_W0, radial_W1)

    ew_pad = jnp.pad(embed_W, ((0, 16 - embed_W.shape[0]), (0, 0)))
    scrow = jnp.pad(scale, (0, 16 - scale.shape[0])).reshape(1, 16)
    shrow = jnp.pad(shift, (0, 16 - shift.shape[0])).reshape(1, 16)
    h0, sclcol, shfcol = _embed(species.reshape(N, 1), ew_pad, scrow, shrow)

    agg0 = _msg_aggregate(h0, w0, iiw, jjw)[:, :N]
    h1 = _update(h0, agg0, msg_W0)

    agg1 = _msg_aggregate(h1, w1, iiw, jjw)[:, :N]

    ro0row = ro0_W.reshape(1, D)
    w1pad = jnp.pad(ro1_W1, ((0, 0), (0, D - MLP_H)))
    w2row = jnp.pad(ro1_W2, ((0, D - MLP_H), (0, 0))).reshape(1, D)
    en = _final(h1, agg1, msg_W1, ro0row, w1pad, w2row, sclcol, shfcol)
    return en[:, 0]


# speedup vs baseline: 2.7441x; 1.2714x over previous
import jax
import jax.numpy as jnp
from jax import lax
from jax.experimental import pallas as pl
from jax.experimental.pallas import tpu as pltpu
from jax.experimental.pallas import tpu_sc as plsc

N = 10000
E = 320000
D = 128
DH = D // 2
NB = 8
RC = 5.0
MLP_H = 64
SILU_SCALE = 1.6765324703310907
PI = 3.141592653589793

NCORE = 2
NSUB = 16
NWORK = NCORE * NSUB
EPW = E // NWORK
EPT = E // NSUB
WB = 80
NWIN = EPT // WB
WROWS = WB * DH // D
NPAD = 10240
ROWS_PER_SUB = NPAD // NSUB
ZR = 128
NZC = ROWS_PER_SUB // ZR

_INTERPRET = False


def _sc_mesh():
    return plsc.VectorSubcoreMesh(core_axis_name="c", subcore_axis_name="s",
                                  num_cores=NCORE, num_subcores=NSUB)


def _geom_body(px_hbm, py_hbm, pz_hbm, sx_hbm, sy_hbm, sz_hbm,
               ii_hbm, jj_hbm, d2_out,
               px, py, pz, sx, sy, sz, iv, jv, d2v):
    cid = lax.axis_index("c")
    sid = lax.axis_index("s")
    wid = sid * NCORE + cid
    base = wid * EPW
    pltpu.sync_copy(px_hbm, px)
    pltpu.sync_copy(py_hbm, py)
    pltpu.sync_copy(pz_hbm, pz)
    pltpu.sync_copy(sx_hbm.at[pl.ds(base, EPW)], sx)
    pltpu.sync_copy(sy_hbm.at[pl.ds(base, EPW)], sy)
    pltpu.sync_copy(sz_hbm.at[pl.ds(base, EPW)], sz)
    pltpu.sync_copy(ii_hbm.at[pl.ds(base, EPW)], iv)
    pltpu.sync_copy(jj_hbm.at[pl.ds(base, EPW)], jv)

    def grp(g, carry):
        s = pl.ds(g * 16, 16)
        a = iv[s]
        b = jv[s]
        dx = plsc.load_gather(px, [a]) - plsc.load_gather(px, [b]) - sx[s]
        dy = plsc.load_gather(py, [a]) - plsc.load_gather(py, [b]) - sy[s]
        dz = plsc.load_gather(pz, [a]) - plsc.load_gather(pz, [b]) - sz[s]
        d2v[s] = dx * dx + dy * dy + dz * dz
        return carry

    lax.fori_loop(0, EPW // 16, grp, 0)
    pltpu.sync_copy(d2v, d2_out.at[pl.ds(base, EPW)])


def _edge_d2(pxyz, sxyz, idx_i, idx_j):
    return pl.kernel(
        _geom_body,
        out_type=jax.ShapeDtypeStruct((E,), jnp.float32),
        mesh=_sc_mesh(),
        scratch_types=[
            pltpu.VMEM((N,), jnp.float32),
            pltpu.VMEM((N,), jnp.float32),
            pltpu.VMEM((N,), jnp.float32),
            pltpu.VMEM((EPW,), jnp.float32),
            pltpu.VMEM((EPW,), jnp.float32),
            pltpu.VMEM((EPW,), jnp.float32),
            pltpu.VMEM((EPW,), jnp.int32),
            pltpu.VMEM((EPW,), jnp.int32),
            pltpu.VMEM((EPW,), jnp.float32),
        ],
        compiler_params=pltpu.CompilerParams(needs_layout_passes=False,
                                             use_tc_tiling_on_sc=False),
        interpret=_INTERPRET,
    )(pxyz[0], pxyz[1], pxyz[2], sxyz[0], sxyz[1], sxyz[2], idx_i, idx_j)


def _msgagg_body(h2flat, w2p, iiw, jjw, agg_out,
                 ii_v, jj_v, jadj, wbuf, hbuf, mbuf, zbuf,
                 wsem, hsem, agg_sh):
    cid = lax.axis_index("c")
    sid = lax.axis_index("s")

    def zrow(e, carry):
        for c in range(DH // 16):
            zbuf[e, pl.ds(c * 16, 16)] = jnp.zeros((16,), jnp.float32)
        return carry

    lax.fori_loop(0, ZR, zrow, 0)
    for k in range(NZC):
        pltpu.sync_copy(zbuf, agg_sh.at[pl.ds(sid * ROWS_PER_SUB + k * ZR, ZR)])
    plsc.subcore_barrier()

    pltpu.sync_copy(iiw.at[sid], ii_v)
    pltpu.sync_copy(jjw.at[sid], jj_v)

    def fill_jadj(w, slot):
        def jrow(q, c2):
            s = pl.ds(q * 16, 16)
            jadj[slot, s] = jj_v[w, s] + cid * N
            return c2
        lax.fori_loop(0, WB // 16, jrow, 0)

    def issue(w, slot):
        row0 = (sid * NWIN + w) * WROWS
        pltpu.async_copy(w2p.at[cid, pl.ds(row0, WROWS)], wbuf.at[slot],
                         wsem.at[slot])
        fill_jadj(w, slot)
        pltpu.async_copy(h2flat.at[jadj.at[slot]], hbuf.at[slot],
                         hsem.at[slot])

    issue(0, 0)

    def win(w, carry):
        slot = lax.rem(w, 2)
        nslot = 1 - slot

        @pl.when(w + 1 < NWIN)
        def _():
            issue(w + 1, nslot)

        row0 = (sid * NWIN + w) * WROWS
        pltpu.make_async_copy(w2p.at[cid, pl.ds(row0, WROWS)],
                              wbuf.at[slot], wsem.at[slot]).wait()
        pltpu.make_async_copy(h2flat.at[jadj.at[slot]],
                              hbuf.at[slot], hsem.at[slot]).wait()

        def epair(p, c2):
            for half in range(2):
                e = p * 2 + half
                for c in range(DH // 16):
                    so = pl.ds(c * 16, 16)
                    si = pl.ds(half * DH + c * 16, 16)
                    mbuf[e, so] = wbuf[slot, p, si] * hbuf[slot, e, so]
            return c2

        lax.fori_loop(0, WB // 2, epair, 0)
        pltpu.sync_copy(mbuf, agg_sh.at[ii_v.at[w]], add=True)
        return carry

    lax.fori_loop(0, NWIN, win, 0)
    plsc.subcore_barrier()

    for k in range(NZC):
        r0 = sid * ROWS_PER_SUB + k * ZR
        pltpu.sync_copy(agg_sh.at[pl.ds(r0, ZR)], zbuf)
        pltpu.sync_copy(zbuf, agg_out.at[cid, pl.ds(r0, ZR)])


def _msg_aggregate(h2flat, w2p, iiw, jjw):
    return pl.kernel(
        _msgagg_body,
        out_type=jax.ShapeDtypeStruct((NCORE, NPAD, DH), jnp.float32),
        mesh=_sc_mesh(),
        scratch_types=[
            pltpu.VMEM((NWIN, WB), jnp.int32),
            pltpu.VMEM((NWIN, WB), jnp.int32),
            pltpu.VMEM((2, WB), jnp.int32),
            pltpu.VMEM((2, WROWS, D), jnp.float32),
            pltpu.VMEM((2, WB, DH), jnp.float32),
            pltpu.VMEM((WB, DH), jnp.float32),
            pltpu.VMEM((ZR, DH), jnp.float32),
            pltpu.SemaphoreType.DMA((2,)),
            pltpu.SemaphoreType.DMA((2,)),
            pltpu.VMEM_SHARED((NPAD, DH), jnp.float32),
        ],
        compiler_params=pltpu.CompilerParams(needs_layout_passes=False,
                                             use_tc_tiling_on_sc=False),
        interpret=_INTERPRET,
    )(h2flat, w2p, iiw, jjw)


BLK_E = 2000
BLK_N = 2000


def _rbfw_body(d2_ref, wd0l_ref, wd0h_ref, wd1l_ref, wd1h_ref,
               w0_ref, w1_ref):
    d2 = d2_ref[0]
    r = jnp.sqrt(d2) + 1e-9
    rinv = 1.0 / r
    nvec = (lax.broadcasted_iota(jnp.int32, (NB, 1), 0).astype(jnp.float32)
            + 1.0) * (PI / RC)
    HB = BLK_E // 2
    rbig = jnp.concatenate(
        (jnp.broadcast_to(r[0:1], (NB, HB)),
         jnp.broadcast_to(r[1:2], (NB, HB))), axis=0)
    ribig = jnp.concatenate(
        (jnp.broadcast_to(rinv[0:1], (NB, HB)),
         jnp.broadcast_to(rinv[1:2], (NB, HB))), axis=0)
    nbig = jnp.concatenate((nvec, nvec), axis=0)
    rbf16 = jnp.sin(nbig * rbig) * ribig
    dn = (((0,), (0,)), ((), ()))
    kw = dict(preferred_element_type=jnp.float32,
              precision=lax.Precision.HIGHEST)
    w0_ref[0] = lax.dot_general(rbf16, wd0l_ref[...], dn, **kw)
    w0_ref[1] = lax.dot_general(rbf16, wd0h_ref[...], dn, **kw)
    w1_ref[0] = lax.dot_general(rbf16, wd1l_ref[...], dn, **kw)
    w1_ref[1] = lax.dot_general(rbf16, wd1h_ref[...], dn, **kw)


def _radial_w(d2m3, wd0l, wd0h, wd1l, wd1h):
    grid = E // BLK_E
    return pl.pallas_call(
        _rbfw_body,
        grid=(grid,),
        in_specs=[
            pl.BlockSpec((1, 2, BLK_E // 2), lambda i: (i, 0, 0)),
            pl.BlockSpec((2 * NB, D), lambda i: (0, 0)),
            pl.BlockSpec((2 * NB, D), lambda i: (0, 0)),
            pl.BlockSpec((2 * NB, D), lambda i: (0, 0)),
            pl.BlockSpec((2 * NB, D), lambda i: (0, 0)),
        ],
        out_specs=[
            pl.BlockSpec((2, BLK_E // 2, D), lambda i: (0, i, 0)),
            pl.BlockSpec((2, BLK_E // 2, D), lambda i: (0, i, 0)),
        ],
        out_shape=[
            jax.ShapeDtypeStruct((2, E * DH // D, D), jnp.float32),
            jax.ShapeDtypeStruct((2, E * DH // D, D), jnp.float32),
        ],
        interpret=_INTERPRET,
    )(d2m3, wd0l, wd0h, wd1l, wd1h)


def _embed_body(spc_ref, ew_ref, scrow_ref, shrow_ref, h2_ref, scl_ref, shf_ref):
    spc = spc_ref[...]
    oh = (spc == lax.broadcasted_iota(jnp.int32, (1, 16), 1)).astype(jnp.float32)
    h = jnp.dot(oh, ew_ref[...], preferred_element_type=jnp.float32,
                precision=lax.Precision.HIGHEST)
    h2_ref[0] = h[:, :DH]
    h2_ref[1] = h[:, DH:]
    scl_ref[...] = jnp.sum(oh * scrow_ref[...], axis=1, keepdims=True)
    shf_ref[...] = jnp.sum(oh * shrow_ref[...], axis=1, keepdims=True)


def _embed(spccol, ew_pad, scrow, shrow):
    grid = N // BLK_N
    return pl.pallas_call(
        _embed_body,
        grid=(grid,),
        in_specs=[
            pl.BlockSpec((BLK_N, 1), lambda i: (i, 0)),
            pl.BlockSpec((16, D), lambda i: (0, 0)),
            pl.BlockSpec((1, 16), lambda i: (0, 0)),
            pl.BlockSpec((1, 16), lambda i: (0, 0)),
        ],
        out_specs=[
            pl.BlockSpec((2, BLK_N, DH), lambda i: (0, i, 0)),
            pl.BlockSpec((BLK_N, 1), lambda i: (i, 0)),
            pl.BlockSpec((BLK_N, 1), lambda i: (i, 0)),
        ],
        out_shape=[
            jax.ShapeDtypeStruct((2, N, DH), jnp.float32),
            jax.ShapeDtypeStruct((N, 1), jnp.float32),
            jax.ShapeDtypeStruct((N, 1), jnp.float32),
        ],
        interpret=_INTERPRET,
    )(spccol, ew_pad, scrow, shrow)


def _upd_body(h2_ref, a_ref, mw_ref, out_ref):
    h = jnp.concatenate((h2_ref[0], h2_ref[1]), axis=1)
    acc = jnp.concatenate((a_ref[0], a_ref[1]), axis=1)
    hn = h + jnp.dot(acc, mw_ref[...], preferred_element_type=jnp.float32,
                     precision=lax.Precision.HIGHEST)
    out_ref[0] = hn[:, :DH]
    out_ref[1] = hn[:, DH:]


def _update(h2, agg, mW):
    grid = N // BLK_N
    return pl.pallas_call(
        _upd_body,
        grid=(grid,),
        in_specs=[
            pl.BlockSpec((2, BLK_N, DH), lambda i: (0, i, 0)),
            pl.BlockSpec((2, BLK_N, DH), lambda i: (0, i, 0)),
            pl.BlockSpec((D, D), lambda i: (0, 0)),
        ],
        out_specs=pl.BlockSpec((2, BLK_N, DH), lambda i: (0, i, 0)),
        out_shape=jax.ShapeDtypeStruct((2, N, DH), jnp.float32),
        interpret=_INTERPRET,
    )(h2, agg, mW)


def _final_body(h2_ref, a_ref, mw_ref, ro0r_ref, w1p_ref, w2r_ref,
                scl_ref, shf_ref, en_ref):
    h1 = jnp.concatenate((h2_ref[0], h2_ref[1]), axis=1)
    acc = jnp.concatenate((a_ref[0], a_ref[1]), axis=1)
    h2 = h1 + jnp.dot(acc, mw_ref[...], preferred_element_type=jnp.float32,
                      precision=lax.Precision.HIGHEST)
    out0 = jnp.sum(h1 * ro0r_ref[...], axis=1, keepdims=True)
    t = jnp.dot(h2, w1p_ref[...], preferred_element_type=jnp.float32,
                precision=lax.Precision.HIGHEST)
    t = (t * jax.nn.sigmoid(t)) * SILU_SCALE
    out1 = jnp.sum(t * w2r_ref[...], axis=1, keepdims=True)
    en_ref[...] = scl_ref[...] * (out0 + out1) + shf_ref[...]


def _final(h2, agg, mW1, ro0row, w1pad, w2row, sclcol, shfcol):
    grid = N // BLK_N
    return pl.pallas_call(
        _final_body,
        grid=(grid,),
        in_specs=[
            pl.BlockSpec((2, BLK_N, DH), lambda i: (0, i, 0)),
            pl.BlockSpec((2, BLK_N, DH), lambda i: (0, i, 0)),
            pl.BlockSpec((D, D), lambda i: (0, 0)),
            pl.BlockSpec((1, D), lambda i: (0, 0)),
            pl.BlockSpec((D, D), lambda i: (0, 0)),
            pl.BlockSpec((1, D), lambda i: (0, 0)),
            pl.BlockSpec((BLK_N, 1), lambda i: (i, 0)),
            pl.BlockSpec((BLK_N, 1), lambda i: (i, 0)),
        ],
        out_specs=pl.BlockSpec((BLK_N, 1), lambda i: (i, 0)),
        out_shape=jax.ShapeDtypeStruct((N, 1), jnp.float32),
        interpret=_INTERPRET,
    )(h2, agg, mW1, ro0row, w1pad, w2row, sclcol, shfcol)


def kernel(positions, shifts, embed_W, radial_W0, radial_W1, msg_W0, msg_W1,
           ro0_W, ro1_W1, ro1_W2, scale, shift, species, edge_index):
    idx_i = edge_index[0]
    idx_j = edge_index[1]
    pxyz = [positions[:, k] for k in range(3)]
    sxyz = [shifts[:, k] for k in range(3)]
    HB = BLK_E // 2
    eord = jnp.arange(E).reshape(E // BLK_E, 2, HB).transpose(0, 2, 1)
    eord = eord.reshape(-1)
    iiw = idx_i[eord].reshape(NSUB, NWIN, WB)
    jjw = idx_j[eord].reshape(NSUB, NWIN, WB)

    def wd_pair(half):
        z = jnp.zeros((NB, DH), jnp.float32)
        top = jnp.concatenate((half, z), axis=1)
        bot = jnp.concatenate((z, half), axis=1)
        return jnp.concatenate((top, bot), axis=0)

    wd0l = wd_pair(radial_W0[:, :DH])
    wd0h = wd_pair(radial_W0[:, DH:])
    wd1l = wd_pair(radial_W1[:, :DH])
    wd1h = wd_pair(radial_W1[:, DH:])

    d2 = _edge_d2(pxyz, sxyz, idx_i, idx_j)
    w0, w1 = _radial_w(d2.reshape(E // BLK_E, 2, HB), wd0l, wd0h, wd1l, wd1h)

    ew_pad = jnp.pad(embed_W, ((0, 16 - embed_W.shape[0]), (0, 0)))
    scrow = jnp.pad(scale, (0, 16 - scale.shape[0])).reshape(1, 16)
    shrow = jnp.pad(shift, (0, 16 - shift.shape[0])).reshape(1, 16)
    h0, sclcol, shfcol = _embed(species.reshape(N, 1), ew_pad, scrow, shrow)

    agg0 = _msg_aggregate(h0.reshape(2 * N, DH), w0, iiw, jjw)[:, :N]
    h1 = _update(h0, agg0, msg_W0)

    agg1 = _msg_aggregate(h1.reshape(2 * N, DH), w1, iiw, jjw)[:, :N]

    ro0row = ro0_W.reshape(1, D)
    w1pad = jnp.pad(ro1_W1, ((0, 0), (0, D - MLP_H)))
    w2row = jnp.pad(ro1_W2, ((0, D - MLP_H), (0, 0))).reshape(1, D)
    en = _final(h1, agg1, msg_W1, ro0row, w1pad, w2row, sclcol, shfcol)
    return en[:, 0]


# speedup vs baseline: 3.8310x; 1.3961x over previous
import jax
import jax.numpy as jnp
from jax import lax
from jax.experimental import pallas as pl
from jax.experimental.pallas import tpu as pltpu
from jax.experimental.pallas import tpu_sc as plsc

N = 10000
E = 320000
D = 128
DH = D // 2
NB = 8
RC = 5.0
MLP_H = 64
SILU_SCALE = 1.6765324703310907
PI = 3.141592653589793

NCORE = 2
NSUB = 16
NWORK = NCORE * NSUB
EPW = E // NWORK
EPT = E // NSUB
WB = 80
NWIN = EPT // WB
WROWS = WB * DH // D
NPAD = 10240
ROWS_PER_SUB = NPAD // NSUB
ZR = 128
NZC = ROWS_PER_SUB // ZR

_INTERPRET = False


def _sc_mesh():
    return plsc.VectorSubcoreMesh(core_axis_name="c", subcore_axis_name="s",
                                  num_cores=NCORE, num_subcores=NSUB)


def _geom_body(px_hbm, py_hbm, pz_hbm, sx_hbm, sy_hbm, sz_hbm,
               ii_hbm, jj_hbm, d2_out,
               px, py, pz, sx, sy, sz, iv, jv, d2v):
    cid = lax.axis_index("c")
    sid = lax.axis_index("s")
    wid = sid * NCORE + cid
    base = wid * EPW
    pltpu.sync_copy(px_hbm, px)
    pltpu.sync_copy(py_hbm, py)
    pltpu.sync_copy(pz_hbm, pz)
    pltpu.sync_copy(sx_hbm.at[pl.ds(base, EPW)], sx)
    pltpu.sync_copy(sy_hbm.at[pl.ds(base, EPW)], sy)
    pltpu.sync_copy(sz_hbm.at[pl.ds(base, EPW)], sz)
    pltpu.sync_copy(ii_hbm.at[pl.ds(base, EPW)], iv)
    pltpu.sync_copy(jj_hbm.at[pl.ds(base, EPW)], jv)

    def grp(g, carry):
        s = pl.ds(g * 16, 16)
        a = iv[s]
        b = jv[s]
        dx = plsc.load_gather(px, [a]) - plsc.load_gather(px, [b]) - sx[s]
        dy = plsc.load_gather(py, [a]) - plsc.load_gather(py, [b]) - sy[s]
        dz = plsc.load_gather(pz, [a]) - plsc.load_gather(pz, [b]) - sz[s]
        d2v[s] = dx * dx + dy * dy + dz * dz
        return carry

    lax.fori_loop(0, EPW // 16, grp, 0)
    pltpu.sync_copy(d2v, d2_out.at[pl.ds(base, EPW)])


def _edge_d2(pxyz, sxyz, idx_i, idx_j):
    return pl.kernel(
        _geom_body,
        out_type=jax.ShapeDtypeStruct((E,), jnp.float32),
        mesh=_sc_mesh(),
        scratch_types=[
            pltpu.VMEM((N,), jnp.float32),
            pltpu.VMEM((N,), jnp.float32),
            pltpu.VMEM((N,), jnp.float32),
            pltpu.VMEM((EPW,), jnp.float32),
            pltpu.VMEM((EPW,), jnp.float32),
            pltpu.VMEM((EPW,), jnp.float32),
            pltpu.VMEM((EPW,), jnp.int32),
            pltpu.VMEM((EPW,), jnp.int32),
            pltpu.VMEM((EPW,), jnp.float32),
        ],
        compiler_params=pltpu.CompilerParams(needs_layout_passes=False,
                                             use_tc_tiling_on_sc=False),
        interpret=_INTERPRET,
    )(pxyz[0], pxyz[1], pxyz[2], sxyz[0], sxyz[1], sxyz[2], idx_i, idx_j)


def _msgagg_body(h2flat, w2p, iiw, jjw, agg_out,
                 ii_v, jj_v, jadj, wbuf, hbuf, mbuf, zbuf,
                 wsem, hsem, ssem, agg_sh):
    cid = lax.axis_index("c")
    sid = lax.axis_index("s")

    def zrow(e, carry):
        for c in range(DH // 16):
            zbuf[e, pl.ds(c * 16, 16)] = jnp.zeros((16,), jnp.float32)
        return carry

    lax.fori_loop(0, ZR, zrow, 0)
    for k in range(NZC):
        pltpu.sync_copy(zbuf, agg_sh.at[pl.ds(sid * ROWS_PER_SUB + k * ZR, ZR)])
    plsc.subcore_barrier()

    pltpu.sync_copy(iiw.at[sid], ii_v)
    pltpu.sync_copy(jjw.at[sid], jj_v)

    def fill_jadj(w, slot):
        def jrow(q, c2):
            s = pl.ds(q * 16, 16)
            jadj[slot, s] = jj_v[w, s] + cid * N
            return c2
        lax.fori_loop(0, WB // 16, jrow, 0)

    def issue(w, slot):
        row0 = (sid * NWIN + w) * WROWS
        pltpu.async_copy(w2p.at[cid, pl.ds(row0, WROWS)], wbuf.at[slot],
                         wsem.at[slot])
        fill_jadj(w, slot)
        pltpu.async_copy(h2flat.at[jadj.at[slot]], hbuf.at[slot],
                         hsem.at[slot])

    issue(0, 0)

    def win(w, carry):
        slot = lax.rem(w, 2)
        nslot = 1 - slot

        @pl.when(w + 1 < NWIN)
        def _():
            issue(w + 1, nslot)

        row0 = (sid * NWIN + w) * WROWS
        pltpu.make_async_copy(w2p.at[cid, pl.ds(row0, WROWS)],
                              wbuf.at[slot], wsem.at[slot]).wait()
        pltpu.make_async_copy(h2flat.at[jadj.at[slot]],
                              hbuf.at[slot], hsem.at[slot]).wait()

        @pl.when(w >= 2)
        def _():
            pltpu.make_async_copy(mbuf.at[slot], agg_sh.at[ii_v.at[w - 2]],
                                  ssem.at[slot]).wait()

        def epair(p, c2):
            for half in range(2):
                e = p * 2 + half
                for c in range(DH // 16):
                    so = pl.ds(c * 16, 16)
                    si = pl.ds(half * DH + c * 16, 16)
                    mbuf[slot, e, so] = wbuf[slot, p, si] * hbuf[slot, e, so]
            return c2

        lax.fori_loop(0, WB // 2, epair, 0)
        pltpu.async_copy(mbuf.at[slot], agg_sh.at[ii_v.at[w]],
                         ssem.at[slot], add=True)
        return carry

    lax.fori_loop(0, NWIN, win, 0)
    for t in (NWIN - 2, NWIN - 1):
        pltpu.make_async_copy(mbuf.at[t % 2], agg_sh.at[ii_v.at[t]],
                              ssem.at[t % 2]).wait()
    plsc.subcore_barrier()

    for k in range(NZC):
        r0 = sid * ROWS_PER_SUB + k * ZR
        pltpu.sync_copy(agg_sh.at[pl.ds(r0, ZR)], zbuf)
        pltpu.sync_copy(zbuf, agg_out.at[cid, pl.ds(r0, ZR)])


def _msg_aggregate(h2flat, w2p, iiw, jjw):
    return pl.kernel(
        _msgagg_body,
        out_type=jax.ShapeDtypeStruct((NCORE, NPAD, DH), jnp.float32),
        mesh=_sc_mesh(),
        scratch_types=[
            pltpu.VMEM((NWIN, WB), jnp.int32),
            pltpu.VMEM((NWIN, WB), jnp.int32),
            pltpu.VMEM((2, WB), jnp.int32),
            pltpu.VMEM((2, WROWS, D), jnp.float32),
            pltpu.VMEM((2, WB, DH), jnp.float32),
            pltpu.VMEM((2, WB, DH), jnp.float32),
            pltpu.VMEM((ZR, DH), jnp.float32),
            pltpu.SemaphoreType.DMA((2,)),
            pltpu.SemaphoreType.DMA((2,)),
            pltpu.SemaphoreType.DMA((2,)),
            pltpu.VMEM_SHARED((NPAD, DH), jnp.float32),
        ],
        compiler_params=pltpu.CompilerParams(needs_layout_passes=False,
                                             use_tc_tiling_on_sc=False),
        interpret=_INTERPRET,
    )(h2flat, w2p, iiw, jjw)


BLK_E = 2000
BLK_N = 2000


NCHK = 4
BLK_R = NCHK * (BLK_E // 2)


def _rbfw_body(d2_ref, wd0l_ref, wd0h_ref, wd1l_ref, wd1h_ref,
               w0_ref, w1_ref):
    d2 = d2_ref[0]
    r = jnp.sqrt(d2) + 1e-9
    rinv = 1.0 / r
    nvec = (lax.broadcasted_iota(jnp.int32, (NB, 1), 0).astype(jnp.float32)
            + 1.0) * (PI / RC)
    HB = BLK_E // 2
    dn = (((0,), (0,)), ((), ()))
    kw = dict(preferred_element_type=jnp.float32,
              precision=lax.Precision.DEFAULT)
    for k in range(NCHK):
        rbig = jnp.concatenate(
            (jnp.broadcast_to(r[2 * k:2 * k + 1], (NB, HB)),
             jnp.broadcast_to(r[2 * k + 1:2 * k + 2], (NB, HB))), axis=0)
        ribig = jnp.concatenate(
            (jnp.broadcast_to(rinv[2 * k:2 * k + 1], (NB, HB)),
             jnp.broadcast_to(rinv[2 * k + 1:2 * k + 2], (NB, HB))), axis=0)
        nbig = jnp.concatenate((nvec, nvec), axis=0)
        rbf16 = jnp.sin(nbig * rbig) * ribig
        rows = slice(k * HB, (k + 1) * HB)
        w0_ref[0, rows] = lax.dot_general(rbf16, wd0l_ref[...], dn, **kw)
        w0_ref[1, rows] = lax.dot_general(rbf16, wd0h_ref[...], dn, **kw)
        w1_ref[0, rows] = lax.dot_general(rbf16, wd1l_ref[...], dn, **kw)
        w1_ref[1, rows] = lax.dot_general(rbf16, wd1h_ref[...], dn, **kw)


def _radial_w(d2m3, wd0l, wd0h, wd1l, wd1h):
    grid = E // (BLK_E * NCHK)
    return pl.pallas_call(
        _rbfw_body,
        grid=(grid,),
        in_specs=[
            pl.BlockSpec((1, 2 * NCHK, BLK_E // 2), lambda i: (i, 0, 0)),
            pl.BlockSpec((2 * NB, D), lambda i: (0, 0)),
            pl.BlockSpec((2 * NB, D), lambda i: (0, 0)),
            pl.BlockSpec((2 * NB, D), lambda i: (0, 0)),
            pl.BlockSpec((2 * NB, D), lambda i: (0, 0)),
        ],
        out_specs=[
            pl.BlockSpec((2, BLK_R, D), lambda i: (0, i, 0)),
            pl.BlockSpec((2, BLK_R, D), lambda i: (0, i, 0)),
        ],
        out_shape=[
            jax.ShapeDtypeStruct((2, E * DH // D, D), jnp.float32),
            jax.ShapeDtypeStruct((2, E * DH // D, D), jnp.float32),
        ],
        interpret=_INTERPRET,
    )(d2m3, wd0l, wd0h, wd1l, wd1h)


def _embed_body(spc_ref, ew_ref, scrow_ref, shrow_ref, h2_ref, scl_ref, shf_ref):
    spc = spc_ref[...]
    oh = (spc == lax.broadcasted_iota(jnp.int32, (1, 16), 1)).astype(jnp.float32)
    h = jnp.dot(oh, ew_ref[...], preferred_element_type=jnp.float32,
                precision=lax.Precision.HIGHEST)
    h2_ref[0] = h[:, :DH]
    h2_ref[1] = h[:, DH:]
    scl_ref[...] = jnp.sum(oh * scrow_ref[...], axis=1, keepdims=True)
    shf_ref[...] = jnp.sum(oh * shrow_ref[...], axis=1, keepdims=True)


def _embed(spccol, ew_pad, scrow, shrow):
    grid = N // BLK_N
    return pl.pallas_call(
        _embed_body,
        grid=(grid,),
        in_specs=[
            pl.BlockSpec((BLK_N, 1), lambda i: (i, 0)),
            pl.BlockSpec((16, D), lambda i: (0, 0)),
            pl.BlockSpec((1, 16), lambda i: (0, 0)),
            pl.BlockSpec((1, 16), lambda i: (0, 0)),
        ],
        out_specs=[
            pl.BlockSpec((2, BLK_N, DH), lambda i: (0, i, 0)),
            pl.BlockSpec((BLK_N, 1), lambda i: (i, 0)),
            pl.BlockSpec((BLK_N, 1), lambda i: (i, 0)),
        ],
        out_shape=[
            jax.ShapeDtypeStruct((2, N, DH), jnp.float32),
            jax.ShapeDtypeStruct((N, 1), jnp.float32),
            jax.ShapeDtypeStruct((N, 1), jnp.float32),
        ],
        interpret=_INTERPRET,
    )(spccol, ew_pad, scrow, shrow)


def _upd_body(h2_ref, a_ref, mw_ref, out_ref):
    h = jnp.concatenate((h2_ref[0], h2_ref[1]), axis=1)
    acc = jnp.concatenate((a_ref[0], a_ref[1]), axis=1)
    hn = h + jnp.dot(acc, mw_ref[...], preferred_element_type=jnp.float32,
                     precision=lax.Precision.HIGHEST)
    out_ref[0] = hn[:, :DH]
    out_ref[1] = hn[:, DH:]


def _update(h2, agg, mW):
    grid = N // BLK_N
    return pl.pallas_call(
        _upd_body,
        grid=(grid,),
        in_specs=[
            pl.BlockSpec((2, BLK_N, DH), lambda i: (0, i, 0)),
            pl.BlockSpec((2, BLK_N, DH), lambda i: (0, i, 0)),
            pl.BlockSpec((D, D), lambda i: (0, 0)),
        ],
        out_specs=pl.BlockSpec((2, BLK_N, DH), lambda i: (0, i, 0)),
        out_shape=jax.ShapeDtypeStruct((2, N, DH), jnp.float32),
        interpret=_INTERPRET,
    )(h2, agg, mW)


def _final_body(h2_ref, a_ref, mw_ref, ro0r_ref, w1p_ref, w2r_ref,
                scl_ref, shf_ref, en_ref):
    h1 = jnp.concatenate((h2_ref[0], h2_ref[1]), axis=1)
    acc = jnp.concatenate((a_ref[0], a_ref[1]), axis=1)
    h2 = h1 + jnp.dot(acc, mw_ref[...], preferred_element_type=jnp.float32,
                      precision=lax.Precision.HIGHEST)
    out0 = jnp.sum(h1 * ro0r_ref[...], axis=1, keepdims=True)
    t = jnp.dot(h2, w1p_ref[...], preferred_element_type=jnp.float32,
                precision=lax.Precision.HIGHEST)
    t = (t * jax.nn.sigmoid(t)) * SILU_SCALE
    out1 = jnp.sum(t * w2r_ref[...], axis=1, keepdims=True)
    en_ref[...] = scl_ref[...] * (out0 + out1) + shf_ref[...]


def _final(h2, agg, mW1, ro0row, w1pad, w2row, sclcol, shfcol):
    grid = N // BLK_N
    return pl.pallas_call(
        _final_body,
        grid=(grid,),
        in_specs=[
            pl.BlockSpec((2, BLK_N, DH), lambda i: (0, i, 0)),
            pl.BlockSpec((2, BLK_N, DH), lambda i: (0, i, 0)),
            pl.BlockSpec((D, D), lambda i: (0, 0)),
            pl.BlockSpec((1, D), lambda i: (0, 0)),
            pl.BlockSpec((D, D), lambda i: (0, 0)),
            pl.BlockSpec((1, D), lambda i: (0, 0)),
            pl.BlockSpec((BLK_N, 1), lambda i: (i, 0)),
            pl.BlockSpec((BLK_N, 1), lambda i: (i, 0)),
        ],
        out_specs=pl.BlockSpec((BLK_N, 1), lambda i: (i, 0)),
        out_shape=jax.ShapeDtypeStruct((N, 1), jnp.float32),
        interpret=_INTERPRET,
    )(h2, agg, mW1, ro0row, w1pad, w2row, sclcol, shfcol)


def kernel(positions, shifts, embed_W, radial_W0, radial_W1, msg_W0, msg_W1,
           ro0_W, ro1_W1, ro1_W2, scale, shift, species, edge_index):
    idx_i = edge_index[0]
    idx_j = edge_index[1]
    pxyz = [positions[:, k] for k in range(3)]
    sxyz = [shifts[:, k] for k in range(3)]
    HB = BLK_E // 2
    eord = jnp.arange(E).reshape(E // BLK_E, 2, HB).transpose(0, 2, 1)
    eord = eord.reshape(-1)
    iiw = idx_i[eord].reshape(NSUB, NWIN, WB)
    jjw = idx_j[eord].reshape(NSUB, NWIN, WB)

    def wd_pair(half):
        z = jnp.zeros((NB, DH), jnp.float32)
        top = jnp.concatenate((half, z), axis=1)
        bot = jnp.concatenate((z, half), axis=1)
        return jnp.concatenate((top, bot), axis=0)

    wd0l = wd_pair(radial_W0[:, :DH])
    wd0h = wd_pair(radial_W0[:, DH:])
    wd1l = wd_pair(radial_W1[:, :DH])
    wd1h = wd_pair(radial_W1[:, DH:])

    d2 = _edge_d2(pxyz, sxyz, idx_i, idx_j)
    w0, w1 = _radial_w(d2.reshape(E // (BLK_E * NCHK), 2 * NCHK, HB),
                       wd0l, wd0h, wd1l, wd1h)

    ew_pad = jnp.pad(embed_W, ((0, 16 - embed_W.shape[0]), (0, 0)))
    scrow = jnp.pad(scale, (0, 16 - scale.shape[0])).reshape(1, 16)
    shrow = jnp.pad(shift, (0, 16 - shift.shape[0])).reshape(1, 16)
    h0, sclcol, shfcol = _embed(species.reshape(N, 1), ew_pad, scrow, shrow)

    agg0 = _msg_aggregate(h0.reshape(2 * N, DH), w0, iiw, jjw)[:, :N]
    h1 = _update(h0, agg0, msg_W0)

    agg1 = _msg_aggregate(h1.reshape(2 * N, DH), w1, iiw, jjw)[:, :N]

    ro0row = ro0_W.reshape(1, D)
    w1pad = jnp.pad(ro1_W1, ((0, 0), (0, D - MLP_H)))
    w2row = jnp.pad(ro1_W2, ((0, D - MLP_H), (0, 0))).reshape(1, D)
    en = _final(h1, agg1, msg_W1, ro0row, w1pad, w2row, sclcol, shfcol)
    return en[:, 0]


# speedup vs baseline: 4.3788x; 1.1430x over previous
import jax
import jax.numpy as jnp
from jax import lax
from jax.experimental import pallas as pl
from jax.experimental.pallas import tpu as pltpu
from jax.experimental.pallas import tpu_sc as plsc

N = 10000
E = 320000
D = 128
DH = D // 2
NB = 8
RC = 5.0
MLP_H = 64
SILU_SCALE = 1.6765324703310907
PI = 3.141592653589793

NCORE = 2
NSUB = 16
NWORK = NCORE * NSUB
EPW = E // NWORK
EPT = E // NSUB
WB = 80
NWIN = EPT // WB
WROWS = WB * DH // D
NBUF = 3
NPAD = 10240
ROWS_PER_SUB = NPAD // NSUB
ZR = 32
NZC = ROWS_PER_SUB // ZR

_INTERPRET = False


def _sc_mesh():
    return plsc.VectorSubcoreMesh(core_axis_name="c", subcore_axis_name="s",
                                  num_cores=NCORE, num_subcores=NSUB)


def _geom_body(px_hbm, py_hbm, pz_hbm, sx_hbm, sy_hbm, sz_hbm,
               ii_hbm, jj_hbm, d2_out, iip_out, jjp_out,
               px, py, pz, sx, sy, sz, iv, jv, d2v, ipv, jpv):
    cid = lax.axis_index("c")
    sid = lax.axis_index("s")
    wid = sid * NCORE + cid
    base = wid * EPW
    pltpu.sync_copy(px_hbm, px)
    pltpu.sync_copy(py_hbm, py)
    pltpu.sync_copy(pz_hbm, pz)
    pltpu.sync_copy(sx_hbm.at[pl.ds(base, EPW)], sx)
    pltpu.sync_copy(sy_hbm.at[pl.ds(base, EPW)], sy)
    pltpu.sync_copy(sz_hbm.at[pl.ds(base, EPW)], sz)
    pltpu.sync_copy(ii_hbm.at[pl.ds(base, EPW)], iv)
    pltpu.sync_copy(jj_hbm.at[pl.ds(base, EPW)], jv)
    lanes = lax.iota(jnp.int32, 16)
    HBE = BLK_E // 2

    def grp(g, carry):
        s = pl.ds(g * 16, 16)
        a = iv[s]
        b = jv[s]
        dx = plsc.load_gather(px, [a]) - plsc.load_gather(px, [b]) - sx[s]
        dy = plsc.load_gather(py, [a]) - plsc.load_gather(py, [b]) - sy[s]
        dz = plsc.load_gather(pz, [a]) - plsc.load_gather(pz, [b]) - sz[s]
        d2v[s] = dx * dx + dy * dy + dz * dz
        pos = g * 16 + lanes
        chunk = pos // BLK_E
        posc = pos - chunk * BLK_E
        src = chunk * BLK_E + (posc >> 1) + (posc & 1) * HBE
        ipv[s] = plsc.load_gather(iv, [src])
        jpv[s] = plsc.load_gather(jv, [src])
        return carry

    lax.fori_loop(0, EPW // 16, grp, 0)
    pltpu.sync_copy(d2v, d2_out.at[pl.ds(base, EPW)])
    pltpu.sync_copy(ipv, iip_out.at[pl.ds(base, EPW)])
    pltpu.sync_copy(jpv, jjp_out.at[pl.ds(base, EPW)])


def _edge_d2(pxyz, sxyz, idx_i, idx_j):
    return pl.kernel(
        _geom_body,
        out_type=(jax.ShapeDtypeStruct((E,), jnp.float32),
                  jax.ShapeDtypeStruct((E,), jnp.int32),
                  jax.ShapeDtypeStruct((E,), jnp.int32)),
        mesh=_sc_mesh(),
        scratch_types=[
            pltpu.VMEM((N,), jnp.float32),
            pltpu.VMEM((N,), jnp.float32),
            pltpu.VMEM((N,), jnp.float32),
            pltpu.VMEM((EPW,), jnp.float32),
            pltpu.VMEM((EPW,), jnp.float32),
            pltpu.VMEM((EPW,), jnp.float32),
            pltpu.VMEM((EPW,), jnp.int32),
            pltpu.VMEM((EPW,), jnp.int32),
            pltpu.VMEM((EPW,), jnp.float32),
            pltpu.VMEM((EPW,), jnp.int32),
            pltpu.VMEM((EPW,), jnp.int32),
        ],
        compiler_params=pltpu.CompilerParams(needs_layout_passes=False,
                                             use_tc_tiling_on_sc=False),
        interpret=_INTERPRET,
    )(pxyz[0], pxyz[1], pxyz[2], sxyz[0], sxyz[1], sxyz[2], idx_i, idx_j)


def _msgagg_body(h2flat, w2p, iiw, jjw, agg_out,
                 ii_v, jj_v, jadj, wbuf, hbuf, mbuf, zbuf,
                 wsem, hsem, ssem, agg_sh):
    cid = lax.axis_index("c")
    sid = lax.axis_index("s")

    def zrow(e, carry):
        for c in range(DH // 16):
            zbuf[e, pl.ds(c * 16, 16)] = jnp.zeros((16,), jnp.float32)
        return carry

    lax.fori_loop(0, ZR, zrow, 0)
    for k in range(NZC):
        pltpu.sync_copy(zbuf, agg_sh.at[pl.ds(sid * ROWS_PER_SUB + k * ZR, ZR)])
    plsc.subcore_barrier()

    pltpu.sync_copy(iiw.at[sid], ii_v)
    pltpu.sync_copy(jjw.at[sid], jj_v)

    def fill_jadj(w, slot):
        def jrow(q, c2):
            s = pl.ds(q * 16, 16)
            jadj[slot, s] = jj_v[w, s] + cid * N
            return c2
        lax.fori_loop(0, WB // 16, jrow, 0)

    def issue(w, slot):
        row0 = (sid * NWIN + w) * WROWS
        pltpu.async_copy(w2p.at[cid, pl.ds(row0, WROWS)], wbuf.at[slot],
                         wsem.at[slot])
        fill_jadj(w, slot)
        pltpu.async_copy(h2flat.at[jadj.at[slot]], hbuf.at[slot],
                         hsem.at[slot])

    for t in range(NBUF - 1):
        issue(t, t)

    def win(w, carry):
        slot = lax.rem(w, NBUF)

        @pl.when(w + NBUF - 1 < NWIN)
        def _():
            issue(w + NBUF - 1, lax.rem(w + NBUF - 1, NBUF))

        row0 = (sid * NWIN + w) * WROWS
        pltpu.make_async_copy(w2p.at[cid, pl.ds(row0, WROWS)],
                              wbuf.at[slot], wsem.at[slot]).wait()
        pltpu.make_async_copy(h2flat.at[jadj.at[slot]],
                              hbuf.at[slot], hsem.at[slot]).wait()

        @pl.when(w >= NBUF)
        def _():
            pltpu.make_async_copy(mbuf.at[slot], agg_sh.at[ii_v.at[w - NBUF]],
                                  ssem.at[slot]).wait()

        def epair(p, c2):
            for half in range(2):
                e = p * 2 + half
                for c in range(DH // 16):
                    so = pl.ds(c * 16, 16)
                    si = pl.ds(half * DH + c * 16, 16)
                    mbuf[slot, e, so] = wbuf[slot, p, si] * hbuf[slot, e, so]
            return c2

        lax.fori_loop(0, WB // 2, epair, 0)
        pltpu.async_copy(mbuf.at[slot], agg_sh.at[ii_v.at[w]],
                         ssem.at[slot], add=True)
        return carry

    lax.fori_loop(0, NWIN, win, 0)
    for t in range(NWIN - NBUF, NWIN):
        pltpu.make_async_copy(mbuf.at[t % NBUF], agg_sh.at[ii_v.at[t]],
                              ssem.at[t % NBUF]).wait()
    plsc.subcore_barrier()

    for k in range(NZC):
        r0 = sid * ROWS_PER_SUB + k * ZR
        pltpu.sync_copy(agg_sh.at[pl.ds(r0, ZR)], zbuf)
        pltpu.sync_copy(zbuf, agg_out.at[cid, pl.ds(r0, ZR)])


def _msg_aggregate(h2flat, w2p, iiw, jjw):
    return pl.kernel(
        _msgagg_body,
        out_type=jax.ShapeDtypeStruct((NCORE, NPAD, DH), jnp.float32),
        mesh=_sc_mesh(),
        scratch_types=[
            pltpu.VMEM((NWIN, WB), jnp.int32),
            pltpu.VMEM((NWIN, WB), jnp.int32),
            pltpu.VMEM((NBUF, WB), jnp.int32),
            pltpu.VMEM((NBUF, WROWS, D), jnp.float32),
            pltpu.VMEM((NBUF, WB, DH), jnp.float32),
            pltpu.VMEM((NBUF, WB, DH), jnp.float32),
            pltpu.VMEM((ZR, DH), jnp.float32),
            pltpu.SemaphoreType.DMA((NBUF,)),
            pltpu.SemaphoreType.DMA((NBUF,)),
            pltpu.SemaphoreType.DMA((NBUF,)),
            pltpu.VMEM_SHARED((NPAD, DH), jnp.float32),
        ],
        compiler_params=pltpu.CompilerParams(needs_layout_passes=False,
                                             use_tc_tiling_on_sc=False),
        interpret=_INTERPRET,
    )(h2flat, w2p, iiw, jjw)


BLK_E = 2000
BLK_N = 2000


NCHK = 4
BLK_R = NCHK * (BLK_E // 2)


def _rbfw_body(d2_ref, wd0l_ref, wd0h_ref, wd1l_ref, wd1h_ref,
               w0_ref, w1_ref):
    d2 = d2_ref[0]
    r = jnp.sqrt(d2) + 1e-9
    rinv = 1.0 / r
    nvec = (lax.broadcasted_iota(jnp.int32, (NB, 1), 0).astype(jnp.float32)
            + 1.0) * (PI / RC)
    HB = BLK_E // 2
    dn = (((0,), (0,)), ((), ()))
    kw = dict(preferred_element_type=jnp.float32,
              precision=lax.Precision.DEFAULT)
    for k in range(NCHK):
        rbig = jnp.concatenate(
            (jnp.broadcast_to(r[2 * k:2 * k + 1], (NB, HB)),
             jnp.broadcast_to(r[2 * k + 1:2 * k + 2], (NB, HB))), axis=0)
        ribig = jnp.concatenate(
            (jnp.broadcast_to(rinv[2 * k:2 * k + 1], (NB, HB)),
             jnp.broadcast_to(rinv[2 * k + 1:2 * k + 2], (NB, HB))), axis=0)
        nbig = jnp.concatenate((nvec, nvec), axis=0)
        rbf16 = jnp.sin(nbig * rbig) * ribig
        rows = slice(k * HB, (k + 1) * HB)
        w0_ref[0, rows] = lax.dot_general(rbf16, wd0l_ref[...], dn, **kw)
        w0_ref[1, rows] = lax.dot_general(rbf16, wd0h_ref[...], dn, **kw)
        w1_ref[0, rows] = lax.dot_general(rbf16, wd1l_ref[...], dn, **kw)
        w1_ref[1, rows] = lax.dot_general(rbf16, wd1h_ref[...], dn, **kw)


def _radial_w(d2m3, wd0l, wd0h, wd1l, wd1h):
    grid = E // (BLK_E * NCHK)
    return pl.pallas_call(
        _rbfw_body,
        grid=(grid,),
        in_specs=[
            pl.BlockSpec((1, 2 * NCHK, BLK_E // 2), lambda i: (i, 0, 0)),
            pl.BlockSpec((2 * NB, D), lambda i: (0, 0)),
            pl.BlockSpec((2 * NB, D), lambda i: (0, 0)),
            pl.BlockSpec((2 * NB, D), lambda i: (0, 0)),
            pl.BlockSpec((2 * NB, D), lambda i: (0, 0)),
        ],
        out_specs=[
            pl.BlockSpec((2, BLK_R, D), lambda i: (0, i, 0)),
            pl.BlockSpec((2, BLK_R, D), lambda i: (0, i, 0)),
        ],
        out_shape=[
            jax.ShapeDtypeStruct((2, E * DH // D, D), jnp.float32),
            jax.ShapeDtypeStruct((2, E * DH // D, D), jnp.float32),
        ],
        interpret=_INTERPRET,
    )(d2m3, wd0l, wd0h, wd1l, wd1h)


def _embed_body(spc_ref, ew_ref, scrow_ref, shrow_ref, h2_ref, scl_ref, shf_ref):
    spc = spc_ref[...]
    oh = (spc == lax.broadcasted_iota(jnp.int32, (1, 16), 1)).astype(jnp.float32)
    h = jnp.dot(oh, ew_ref[...], preferred_element_type=jnp.float32,
                precision=lax.Precision.HIGHEST)
    h2_ref[0] = h[:, :DH]
    h2_ref[1] = h[:, DH:]
    scl_ref[...] = jnp.sum(oh * scrow_ref[...], axis=1, keepdims=True)
    shf_ref[...] = jnp.sum(oh * shrow_ref[...], axis=1, keepdims=True)


def _embed(spccol, ew_pad, scrow, shrow):
    grid = N // BLK_N
    return pl.pallas_call(
        _embed_body,
        grid=(grid,),
        in_specs=[
            pl.BlockSpec((BLK_N, 1), lambda i: (i, 0)),
            pl.BlockSpec((16, D), lambda i: (0, 0)),
            pl.BlockSpec((1, 16), lambda i: (0, 0)),
            pl.BlockSpec((1, 16), lambda i: (0, 0)),
        ],
        out_specs=[
            pl.BlockSpec((2, BLK_N, DH), lambda i: (0, i, 0)),
            pl.BlockSpec((BLK_N, 1), lambda i: (i, 0)),
            pl.BlockSpec((BLK_N, 1), lambda i: (i, 0)),
        ],
        out_shape=[
            jax.ShapeDtypeStruct((2, N, DH), jnp.float32),
            jax.ShapeDtypeStruct((N, 1), jnp.float32),
            jax.ShapeDtypeStruct((N, 1), jnp.float32),
        ],
        interpret=_INTERPRET,
    )(spccol, ew_pad, scrow, shrow)


def _upd_body(h2_ref, a_ref, mw_ref, out_ref):
    h = jnp.concatenate((h2_ref[0], h2_ref[1]), axis=1)
    acc = jnp.concatenate((a_ref[0], a_ref[1]), axis=1)
    hn = h + jnp.dot(acc, mw_ref[...], preferred_element_type=jnp.float32,
                     precision=lax.Precision.HIGHEST)
    out_ref[0] = hn[:, :DH]
    out_ref[1] = hn[:, DH:]


def _update(h2, agg, mW):
    grid = N // BLK_N
    return pl.pallas_call(
        _upd_body,
        grid=(grid,),
        in_specs=[
            pl.BlockSpec((2, BLK_N, DH), lambda i: (0, i, 0)),
            pl.BlockSpec((2, BLK_N, DH), lambda i: (0, i, 0)),
            pl.BlockSpec((D, D), lambda i: (0, 0)),
        ],
        out_specs=pl.BlockSpec((2, BLK_N, DH), lambda i: (0, i, 0)),
        out_shape=jax.ShapeDtypeStruct((2, N, DH), jnp.float32),
        interpret=_INTERPRET,
    )(h2, agg, mW)


def _final_body(h2_ref, a_ref, mw_ref, ro0r_ref, w1p_ref, w2r_ref,
                scl_ref, shf_ref, en_ref):
    h1 = jnp.concatenate((h2_ref[0], h2_ref[1]), axis=1)
    acc = jnp.concatenate((a_ref[0], a_ref[1]), axis=1)
    h2 = h1 + jnp.dot(acc, mw_ref[...], preferred_element_type=jnp.float32,
                      precision=lax.Precision.HIGHEST)
    out0 = jnp.sum(h1 * ro0r_ref[...], axis=1, keepdims=True)
    t = jnp.dot(h2, w1p_ref[...], preferred_element_type=jnp.float32,
                precision=lax.Precision.HIGHEST)
    t = (t * jax.nn.sigmoid(t)) * SILU_SCALE
    out1 = jnp.sum(t * w2r_ref[...], axis=1, keepdims=True)
    en_ref[...] = scl_ref[...] * (out0 + out1) + shf_ref[...]


def _final(h2, agg, mW1, ro0row, w1pad, w2row, sclcol, shfcol):
    grid = N // BLK_N
    return pl.pallas_call(
        _final_body,
        grid=(grid,),
        in_specs=[
            pl.BlockSpec((2, BLK_N, DH), lambda i: (0, i, 0)),
            pl.BlockSpec((2, BLK_N, DH), lambda i: (0, i, 0)),
            pl.BlockSpec((D, D), lambda i: (0, 0)),
            pl.BlockSpec((1, D), lambda i: (0, 0)),
            pl.BlockSpec((D, D), lambda i: (0, 0)),
            pl.BlockSpec((1, D), lambda i: (0, 0)),
            pl.BlockSpec((BLK_N, 1), lambda i: (i, 0)),
            pl.BlockSpec((BLK_N, 1), lambda i: (i, 0)),
        ],
        out_specs=pl.BlockSpec((BLK_N, 1), lambda i: (i, 0)),
        out_shape=jax.ShapeDtypeStruct((N, 1), jnp.float32),
        interpret=_INTERPRET,
    )(h2, agg, mW1, ro0row, w1pad, w2row, sclcol, shfcol)


def kernel(positions, shifts, embed_W, radial_W0, radial_W1, msg_W0, msg_W1,
           ro0_W, ro1_W1, ro1_W2, scale, shift, species, edge_index):
    idx_i = edge_index[0]
    idx_j = edge_index[1]
    pxyz = [positions[:, k] for k in range(3)]
    sxyz = [shifts[:, k] for k in range(3)]
    HB = BLK_E // 2

    def wd_pair(half):
        z = jnp.zeros((NB, DH), jnp.float32)
        top = jnp.concatenate((half, z), axis=1)
        bot = jnp.concatenate((z, half), axis=1)
        return jnp.concatenate((top, bot), axis=0)

    wd0l = wd_pair(radial_W0[:, :DH])
    wd0h = wd_pair(radial_W0[:, DH:])
    wd1l = wd_pair(radial_W1[:, :DH])
    wd1h = wd_pair(radial_W1[:, DH:])

    d2, iip, jjp = _edge_d2(pxyz, sxyz, idx_i, idx_j)
    iiw = iip.reshape(NSUB, NWIN, WB)
    jjw = jjp.reshape(NSUB, NWIN, WB)
    w0, w1 = _radial_w(d2.reshape(E // (BLK_E * NCHK), 2 * NCHK, HB),
                       wd0l, wd0h, wd1l, wd1h)

    ew_pad = jnp.pad(embed_W, ((0, 16 - embed_W.shape[0]), (0, 0)))
    scrow = jnp.pad(scale, (0, 16 - scale.shape[0])).reshape(1, 16)
    shrow = jnp.pad(shift, (0, 16 - shift.shape[0])).reshape(1, 16)
    h0, sclcol, shfcol = _embed(species.reshape(N, 1), ew_pad, scrow, shrow)

    agg0 = _msg_aggregate(h0.reshape(2 * N, DH), w0, iiw, jjw)[:, :N]
    h1 = _update(h0, agg0, msg_W0)

    agg1 = _msg_aggregate(h1.reshape(2 * N, DH), w1, iiw, jjw)[:, :N]

    ro0row = ro0_W.reshape(1, D)
    w1pad = jnp.pad(ro1_W1, ((0, 0), (0, D - MLP_H)))
    w2row = jnp.pad(ro1_W2, ((0, D - MLP_H), (0, 0))).reshape(1, D)
    en = _final(h1, agg1, msg_W1, ro0row, w1pad, w2row, sclcol, shfcol)
    return en[:, 0]


# speedup vs baseline: 4.4101x; 1.0071x over previous
import jax
import jax.numpy as jnp
from jax import lax
from jax.experimental import pallas as pl
from jax.experimental.pallas import tpu as pltpu
from jax.experimental.pallas import tpu_sc as plsc

N = 10000
E = 320000
D = 128
DH = D // 2
NB = 8
RC = 5.0
MLP_H = 64
SILU_SCALE = 1.6765324703310907
PI = 3.141592653589793

NCORE = 2
NSUB = 16
NWORK = NCORE * NSUB
EPW = E // NWORK
EPT = E // NSUB
WB = 80
NWIN = EPT // WB
WROWS = WB * DH // D
NBUF = 3
NPAD = 10240
ROWS_PER_SUB = NPAD // NSUB
ZR = 32
NZC = ROWS_PER_SUB // ZR

_INTERPRET = False


def _sc_mesh():
    return plsc.VectorSubcoreMesh(core_axis_name="c", subcore_axis_name="s",
                                  num_cores=NCORE, num_subcores=NSUB)


def _geom_body(px_hbm, py_hbm, pz_hbm, sx_hbm, sy_hbm, sz_hbm,
               ii_hbm, jj_hbm, d2_out, iip_out, jjp_out,
               px, py, pz, sx, sy, sz, iv, jv, d2v, ipv, jpv):
    cid = lax.axis_index("c")
    sid = lax.axis_index("s")
    wid = sid * NCORE + cid
    base = wid * EPW
    pltpu.sync_copy(px_hbm, px)
    pltpu.sync_copy(py_hbm, py)
    pltpu.sync_copy(pz_hbm, pz)
    pltpu.sync_copy(sx_hbm.at[pl.ds(base, EPW)], sx)
    pltpu.sync_copy(sy_hbm.at[pl.ds(base, EPW)], sy)
    pltpu.sync_copy(sz_hbm.at[pl.ds(base, EPW)], sz)
    pltpu.sync_copy(ii_hbm.at[pl.ds(base, EPW)], iv)
    pltpu.sync_copy(jj_hbm.at[pl.ds(base, EPW)], jv)
    lanes = lax.iota(jnp.int32, 16)
    HBE = BLK_E // 2

    def grp(g, carry):
        s = pl.ds(g * 16, 16)
        a = iv[s]
        b = jv[s]
        dx = plsc.load_gather(px, [a]) - plsc.load_gather(px, [b]) - sx[s]
        dy = plsc.load_gather(py, [a]) - plsc.load_gather(py, [b]) - sy[s]
        dz = plsc.load_gather(pz, [a]) - plsc.load_gather(pz, [b]) - sz[s]
        d2v[s] = dx * dx + dy * dy + dz * dz
        pos = g * 16 + lanes
        chunk = pos // BLK_E
        posc = pos - chunk * BLK_E
        src = chunk * BLK_E + (posc >> 1) + (posc & 1) * HBE
        ipv[s] = plsc.load_gather(iv, [src])
        jpv[s] = plsc.load_gather(jv, [src])
        return carry

    lax.fori_loop(0, EPW // 16, grp, 0)
    pltpu.sync_copy(d2v, d2_out.at[pl.ds(base, EPW)])
    pltpu.sync_copy(ipv, iip_out.at[pl.ds(base, EPW)])
    pltpu.sync_copy(jpv, jjp_out.at[pl.ds(base, EPW)])


def _edge_d2(pxyz, sxyz, idx_i, idx_j):
    return pl.kernel(
        _geom_body,
        out_type=(jax.ShapeDtypeStruct((E,), jnp.float32),
                  jax.ShapeDtypeStruct((E,), jnp.int32),
                  jax.ShapeDtypeStruct((E,), jnp.int32)),
        mesh=_sc_mesh(),
        scratch_types=[
            pltpu.VMEM((N,), jnp.float32),
            pltpu.VMEM((N,), jnp.float32),
            pltpu.VMEM((N,), jnp.float32),
            pltpu.VMEM((EPW,), jnp.float32),
            pltpu.VMEM((EPW,), jnp.float32),
            pltpu.VMEM((EPW,), jnp.float32),
            pltpu.VMEM((EPW,), jnp.int32),
            pltpu.VMEM((EPW,), jnp.int32),
            pltpu.VMEM((EPW,), jnp.float32),
            pltpu.VMEM((EPW,), jnp.int32),
            pltpu.VMEM((EPW,), jnp.int32),
        ],
        compiler_params=pltpu.CompilerParams(needs_layout_passes=False,
                                             use_tc_tiling_on_sc=False),
        interpret=_INTERPRET,
    )(pxyz[0], pxyz[1], pxyz[2], sxyz[0], sxyz[1], sxyz[2], idx_i, idx_j)


def _msgagg_body(h2flat, w2p, iiw, jjw, agg_out,
                 ii_v, jj_v, wbuf, hbuf, mbuf, zbuf,
                 wsem, hsem, ssem, agg_sh):
    cid = lax.axis_index("c")
    sid = lax.axis_index("s")

    def zrow(e, carry):
        for c in range(DH // 16):
            zbuf[e, pl.ds(c * 16, 16)] = jnp.zeros((16,), jnp.float32)
        return carry

    lax.fori_loop(0, ZR, zrow, 0)
    for k in range(NZC):
        pltpu.sync_copy(zbuf, agg_sh.at[pl.ds(sid * ROWS_PER_SUB + k * ZR, ZR)])
    plsc.subcore_barrier()

    pltpu.sync_copy(iiw.at[sid], ii_v)
    pltpu.sync_copy(jjw.at[sid], jj_v)

    def adj(v, c2):
        def jrow(q, c3):
            s = pl.ds(q * 16, 16)
            jj_v[v, s] = jj_v[v, s] + cid * N
            return c3
        lax.fori_loop(0, WB // 16, jrow, 0)
        return c2

    lax.fori_loop(0, NWIN, adj, 0)

    def issue(w, slot):
        row0 = (sid * NWIN + w) * WROWS
        pltpu.async_copy(w2p.at[cid, pl.ds(row0, WROWS)], wbuf.at[slot],
                         wsem.at[slot])
        pltpu.async_copy(h2flat.at[jj_v.at[w]], hbuf.at[slot],
                         hsem.at[slot])

    for t in range(NBUF - 1):
        issue(t, t)

    def win(w, carry):
        slot = lax.rem(w, NBUF)

        @pl.when(w + NBUF - 1 < NWIN)
        def _():
            issue(w + NBUF - 1, lax.rem(w + NBUF - 1, NBUF))

        row0 = (sid * NWIN + w) * WROWS
        pltpu.make_async_copy(w2p.at[cid, pl.ds(row0, WROWS)],
                              wbuf.at[slot], wsem.at[slot]).wait()
        pltpu.make_async_copy(h2flat.at[jj_v.at[w]],
                              hbuf.at[slot], hsem.at[slot]).wait()

        @pl.when(w >= NBUF)
        def _():
            pltpu.make_async_copy(mbuf.at[slot], agg_sh.at[ii_v.at[w - NBUF]],
                                  ssem.at[slot]).wait()

        def epair(p, c2):
            for half in range(2):
                e = p * 2 + half
                for c in range(DH // 16):
                    so = pl.ds(c * 16, 16)
                    si = pl.ds(half * DH + c * 16, 16)
                    mbuf[slot, e, so] = wbuf[slot, p, si] * hbuf[slot, e, so]
            return c2

        lax.fori_loop(0, WB // 2, epair, 0)
        pltpu.async_copy(mbuf.at[slot], agg_sh.at[ii_v.at[w]],
                         ssem.at[slot], add=True)
        return carry

    lax.fori_loop(0, NWIN, win, 0)
    for t in range(NWIN - NBUF, NWIN):
        pltpu.make_async_copy(mbuf.at[t % NBUF], agg_sh.at[ii_v.at[t]],
                              ssem.at[t % NBUF]).wait()
    plsc.subcore_barrier()

    for k in range(NZC):
        r0 = sid * ROWS_PER_SUB + k * ZR
        pltpu.sync_copy(agg_sh.at[pl.ds(r0, ZR)], zbuf)
        pltpu.sync_copy(zbuf, agg_out.at[cid, pl.ds(r0, ZR)])


def _msg_aggregate(h2flat, w2p, iiw, jjw):
    return pl.kernel(
        _msgagg_body,
        out_type=jax.ShapeDtypeStruct((NCORE, NPAD, DH), jnp.float32),
        mesh=_sc_mesh(),
        scratch_types=[
            pltpu.VMEM((NWIN, WB), jnp.int32),
            pltpu.VMEM((NWIN, WB), jnp.int32),
            pltpu.VMEM((NBUF, WROWS, D), jnp.float32),
            pltpu.VMEM((NBUF, WB, DH), jnp.float32),
            pltpu.VMEM((NBUF, WB, DH), jnp.float32),
            pltpu.VMEM((ZR, DH), jnp.float32),
            pltpu.SemaphoreType.DMA((NBUF,)),
            pltpu.SemaphoreType.DMA((NBUF,)),
            pltpu.SemaphoreType.DMA((NBUF,)),
            pltpu.VMEM_SHARED((NPAD, DH), jnp.float32),
        ],
        compiler_params=pltpu.CompilerParams(needs_layout_passes=False,
                                             use_tc_tiling_on_sc=False),
        interpret=_INTERPRET,
    )(h2flat, w2p, iiw, jjw)


BLK_E = 2000
BLK_N = 2000


NCHK = 4
BLK_R = NCHK * (BLK_E // 2)


def _rbfw_body(d2_ref, wd0l_ref, wd0h_ref, wd1l_ref, wd1h_ref,
               w0_ref, w1_ref):
    d2 = d2_ref[0]
    r = jnp.sqrt(d2) + 1e-9
    rinv = 1.0 / r
    nvec = (lax.broadcasted_iota(jnp.int32, (NB, 1), 0).astype(jnp.float32)
            + 1.0) * (PI / RC)
    HB = BLK_E // 2
    dn = (((0,), (0,)), ((), ()))
    kw = dict(preferred_element_type=jnp.float32,
              precision=lax.Precision.DEFAULT)
    for k in range(NCHK):
        rbig = jnp.concatenate(
            (jnp.broadcast_to(r[2 * k:2 * k + 1], (NB, HB)),
             jnp.broadcast_to(r[2 * k + 1:2 * k + 2], (NB, HB))), axis=0)
        ribig = jnp.concatenate(
            (jnp.broadcast_to(rinv[2 * k:2 * k + 1], (NB, HB)),
             jnp.broadcast_to(rinv[2 * k + 1:2 * k + 2], (NB, HB))), axis=0)
        nbig = jnp.concatenate((nvec, nvec), axis=0)
        rbf16 = jnp.sin(nbig * rbig) * ribig
        rows = slice(k * HB, (k + 1) * HB)
        w0_ref[0, rows] = lax.dot_general(rbf16, wd0l_ref[...], dn, **kw)
        w0_ref[1, rows] = lax.dot_general(rbf16, wd0h_ref[...], dn, **kw)
        w1_ref[0, rows] = lax.dot_general(rbf16, wd1l_ref[...], dn, **kw)
        w1_ref[1, rows] = lax.dot_general(rbf16, wd1h_ref[...], dn, **kw)


def _radial_w(d2m3, wd0l, wd0h, wd1l, wd1h):
    grid = E // (BLK_E * NCHK)
    return pl.pallas_call(
        _rbfw_body,
        grid=(grid,),
        in_specs=[
            pl.BlockSpec((1, 2 * NCHK, BLK_E // 2), lambda i: (i, 0, 0)),
            pl.BlockSpec((2 * NB, D), lambda i: (0, 0)),
            pl.BlockSpec((2 * NB, D), lambda i: (0, 0)),
            pl.BlockSpec((2 * NB, D), lambda i: (0, 0)),
            pl.BlockSpec((2 * NB, D), lambda i: (0, 0)),
        ],
        out_specs=[
            pl.BlockSpec((2, BLK_R, D), lambda i: (0, i, 0)),
            pl.BlockSpec((2, BLK_R, D), lambda i: (0, i, 0)),
        ],
        out_shape=[
            jax.ShapeDtypeStruct((2, E * DH // D, D), jnp.float32),
            jax.ShapeDtypeStruct((2, E * DH // D, D), jnp.float32),
        ],
        interpret=_INTERPRET,
    )(d2m3, wd0l, wd0h, wd1l, wd1h)


def _embed_body(spc_ref, ew_ref, scrow_ref, shrow_ref, h2_ref, scl_ref, shf_ref):
    spc = spc_ref[...]
    oh = (spc == lax.broadcasted_iota(jnp.int32, (1, 16), 1)).astype(jnp.float32)
    h = jnp.dot(oh, ew_ref[...], preferred_element_type=jnp.float32,
                precision=lax.Precision.HIGHEST)
    h2_ref[0] = h[:, :DH]
    h2_ref[1] = h[:, DH:]
    scl_ref[...] = jnp.sum(oh * scrow_ref[...], axis=1, keepdims=True)
    shf_ref[...] = jnp.sum(oh * shrow_ref[...], axis=1, keepdims=True)


def _embed(spccol, ew_pad, scrow, shrow):
    grid = N // BLK_N
    return pl.pallas_call(
        _embed_body,
        grid=(grid,),
        in_specs=[
            pl.BlockSpec((BLK_N, 1), lambda i: (i, 0)),
            pl.BlockSpec((16, D), lambda i: (0, 0)),
            pl.BlockSpec((1, 16), lambda i: (0, 0)),
            pl.BlockSpec((1, 16), lambda i: (0, 0)),
        ],
        out_specs=[
            pl.BlockSpec((2, BLK_N, DH), lambda i: (0, i, 0)),
            pl.BlockSpec((BLK_N, 1), lambda i: (i, 0)),
            pl.BlockSpec((BLK_N, 1), lambda i: (i, 0)),
        ],
        out_shape=[
            jax.ShapeDtypeStruct((2, N, DH), jnp.float32),
            jax.ShapeDtypeStruct((N, 1), jnp.float32),
            jax.ShapeDtypeStruct((N, 1), jnp.float32),
        ],
        interpret=_INTERPRET,
    )(spccol, ew_pad, scrow, shrow)


def _upd_body(h2_ref, a_ref, mw_ref, out_ref):
    h = jnp.concatenate((h2_ref[0], h2_ref[1]), axis=1)
    acc = jnp.concatenate((a_ref[0], a_ref[1]), axis=1)
    hn = h + jnp.dot(acc, mw_ref[...], preferred_element_type=jnp.float32,
                     precision=lax.Precision.HIGHEST)
    out_ref[0] = hn[:, :DH]
    out_ref[1] = hn[:, DH:]


def _update(h2, agg, mW):
    grid = N // BLK_N
    return pl.pallas_call(
        _upd_body,
        grid=(grid,),
        in_specs=[
            pl.BlockSpec((2, BLK_N, DH), lambda i: (0, i, 0)),
            pl.BlockSpec((2, BLK_N, DH), lambda i: (0, i, 0)),
            pl.BlockSpec((D, D), lambda i: (0, 0)),
        ],
        out_specs=pl.BlockSpec((2, BLK_N, DH), lambda i: (0, i, 0)),
        out_shape=jax.ShapeDtypeStruct((2, N, DH), jnp.float32),
        interpret=_INTERPRET,
    )(h2, agg, mW)


def _final_body(h2_ref, a_ref, mw_ref, ro0r_ref, w1p_ref, w2r_ref,
                scl_ref, shf_ref, en_ref):
    h1 = jnp.concatenate((h2_ref[0], h2_ref[1]), axis=1)
    acc = jnp.concatenate((a_ref[0], a_ref[1]), axis=1)
    h2 = h1 + jnp.dot(acc, mw_ref[...], preferred_element_type=jnp.float32,
                      precision=lax.Precision.HIGHEST)
    out0 = jnp.sum(h1 * ro0r_ref[...], axis=1, keepdims=True)
    t = jnp.dot(h2, w1p_ref[...], preferred_element_type=jnp.float32,
                precision=lax.Precision.HIGHEST)
    t = (t * jax.nn.sigmoid(t)) * SILU_SCALE
    out1 = jnp.sum(t * w2r_ref[...], axis=1, keepdims=True)
    en_ref[...] = scl_ref[...] * (out0 + out1) + shf_ref[...]


def _final(h2, agg, mW1, ro0row, w1pad, w2row, sclcol, shfcol):
    grid = N // BLK_N
    return pl.pallas_call(
        _final_body,
        grid=(grid,),
        in_specs=[
            pl.BlockSpec((2, BLK_N, DH), lambda i: (0, i, 0)),
            pl.BlockSpec((2, BLK_N, DH), lambda i: (0, i, 0)),
            pl.BlockSpec((D, D), lambda i: (0, 0)),
            pl.BlockSpec((1, D), lambda i: (0, 0)),
            pl.BlockSpec((D, D), lambda i: (0, 0)),
            pl.BlockSpec((1, D), lambda i: (0, 0)),
            pl.BlockSpec((BLK_N, 1), lambda i: (i, 0)),
            pl.BlockSpec((BLK_N, 1), lambda i: (i, 0)),
        ],
        out_specs=pl.BlockSpec((BLK_N, 1), lambda i: (i, 0)),
        out_shape=jax.ShapeDtypeStruct((N, 1), jnp.float32),
        interpret=_INTERPRET,
    )(h2, agg, mW1, ro0row, w1pad, w2row, sclcol, shfcol)


def kernel(positions, shifts, embed_W, radial_W0, radial_W1, msg_W0, msg_W1,
           ro0_W, ro1_W1, ro1_W2, scale, shift, species, edge_index):
    idx_i = edge_index[0]
    idx_j = edge_index[1]
    pxyz = [positions[:, k] for k in range(3)]
    sxyz = [shifts[:, k] for k in range(3)]
    HB = BLK_E // 2

    def wd_pair(half):
        z = jnp.zeros((NB, DH), jnp.float32)
        top = jnp.concatenate((half, z), axis=1)
        bot = jnp.concatenate((z, half), axis=1)
        return jnp.concatenate((top, bot), axis=0)

    wd0l = wd_pair(radial_W0[:, :DH])
    wd0h = wd_pair(radial_W0[:, DH:])
    wd1l = wd_pair(radial_W1[:, :DH])
    wd1h = wd_pair(radial_W1[:, DH:])

    d2, iip, jjp = _edge_d2(pxyz, sxyz, idx_i, idx_j)
    iiw = iip.reshape(NSUB, NWIN, WB)
    jjw = jjp.reshape(NSUB, NWIN, WB)
    w0, w1 = _radial_w(d2.reshape(E // (BLK_E * NCHK), 2 * NCHK, HB),
                       wd0l, wd0h, wd1l, wd1h)

    ew_pad = jnp.pad(embed_W, ((0, 16 - embed_W.shape[0]), (0, 0)))
    scrow = jnp.pad(scale, (0, 16 - scale.shape[0])).reshape(1, 16)
    shrow = jnp.pad(shift, (0, 16 - shift.shape[0])).reshape(1, 16)
    h0, sclcol, shfcol = _embed(species.reshape(N, 1), ew_pad, scrow, shrow)

    agg0 = _msg_aggregate(h0.reshape(2 * N, DH), w0, iiw, jjw)[:, :N]
    h1 = _update(h0, agg0, msg_W0)

    agg1 = _msg_aggregate(h1.reshape(2 * N, DH), w1, iiw, jjw)[:, :N]

    ro0row = ro0_W.reshape(1, D)
    w1pad = jnp.pad(ro1_W1, ((0, 0), (0, D - MLP_H)))
    w2row = jnp.pad(ro1_W2, ((0, D - MLP_H), (0, 0))).reshape(1, D)
    en = _final(h1, agg1, msg_W1, ro0row, w1pad, w2row, sclcol, shfcol)
    return en[:, 0]
